# baseline (speedup 1.0000x reference)
import functools

import jax
import jax.numpy as jnp
from jax import lax
from jax.experimental import pallas as pl
from jax.experimental.pallas import tpu as pltpu
from jax.experimental.pallas import tpu_sc as plsc

N = 50000
E = 800000
H = 64
NG = 64
NUM_LAYERS = 3

NCORES = 2
NSUB = 16
HALF = N // NCORES
CHUNK = 512
CROWS = CHUNK // 128
CHUNKS_PER_SUB = 100
EPAD = NSUB * CHUNKS_PER_SUB * CHUNK
ACC_ROWS = 25088
NW = 32
EPW = EPAD // NW
NBUCKET = 4
PGROUP = 1024
SLOTCAP = EPW + PGROUP
STAGECAP = 2 * PGROUP + PGROUP + 16
TRASH_OFF = 3 * PGROUP
NPHASE = 2
QUART = HALF // NPHASE
ACC2_ROWS = 12512
ZROWS = 128
ZPS = ACC2_ROWS // NSUB

BN = 5000
GRID = N // BN
CODES = 104


NSLOT = 2
GROUP = NSLOT * CHUNK
GROUPS_PER_SUB = CHUNKS_PER_SUB // NSLOT
ROWBYTES = CHUNK * H * 4


def _sc_partition_body(src_hbm, dst_hbm, psrc_hbm, pdst_hbm, cnt_hbm, *scr):
    sg, dg = scr[0], scr[1]
    ssts = scr[2:2 + NBUCKET]
    dsts = scr[2 + NBUCKET:2 + 2 * NBUCKET]
    cbuf = scr[2 + 2 * NBUCKET]
    c = lax.axis_index("c")
    s = lax.axis_index("s")
    w = c * NSUB + s
    ebase = w * EPW

    def group_body(g, carry):
        pltpu.sync_copy(src_hbm.at[pl.ds(ebase + g * PGROUP, PGROUP)], sg)
        pltpu.sync_copy(dst_hbm.at[pl.ds(ebase + g * PGROUP, PGROUP)], dg)

        def sub_body(i, inner):
            d = dg[pl.ds(i * 16, 16)]
            v = sg[pl.ds(i * 16, 16)]
            out = []
            lane = lax.iota(jnp.int32, 16)
            for b in range(NBUCKET):
                cnt_b, nf_b = inner[2 * b], inner[2 * b + 1]
                mb = (d >= b * QUART) & (d < (b + 1) * QUART)
                ps = plsc.cumsum(mb.astype(jnp.int32))
                tgt = jnp.where(mb, cnt_b + ps - 1, TRASH_OFF + lane)
                plsc.store_scatter(ssts[b], [tgt], v)
                plsc.store_scatter(dsts[b], [tgt], d)
                cnt_b = cnt_b + ps[15]

                def flush(cn, nf, b=b):
                    pltpu.sync_copy(
                        ssts[b].at[pl.ds(0, 2 * PGROUP)],
                        psrc_hbm.at[b, w, pl.ds(nf * 2 * PGROUP, 2 * PGROUP)])
                    pltpu.sync_copy(
                        dsts[b].at[pl.ds(0, 2 * PGROUP)],
                        pdst_hbm.at[b, w, pl.ds(nf * 2 * PGROUP, 2 * PGROUP)])
                    ssts[b][pl.ds(0, 16)] = ssts[b][pl.ds(2 * PGROUP, 16)]
                    dsts[b][pl.ds(0, 16)] = dsts[b][pl.ds(2 * PGROUP, 16)]
                    return cn - 2 * PGROUP, nf + 1

                cnt_b, nf_b = lax.cond(cnt_b >= 2 * PGROUP, flush,
                                       lambda cn, nf: (cn, nf), cnt_b, nf_b)
                out += [cnt_b, nf_b]
            return tuple(out)

        return lax.fori_loop(0, PGROUP // 16, sub_body, carry)

    state = lax.fori_loop(0, EPW // PGROUP, group_body,
                          tuple([jnp.int32(0)] * (2 * NBUCKET)))

    trash_src = jnp.zeros((16,), jnp.int32)
    trash_dst = jnp.full((16,), -1, jnp.int32)
    for b in range(NBUCKET):
        cnt_b, nf_b = state[2 * b], state[2 * b + 1]
        for t in range(PGROUP // 16):
            ssts[b][pl.ds(cnt_b + t * 16, 16)] = trash_src
            dsts[b][pl.ds(cnt_b + t * 16, 16)] = trash_dst
        ngr = (cnt_b + PGROUP - 1) // PGROUP

        @pl.when(ngr >= 1)
        def _(b=b, nf_b=nf_b):
            pltpu.sync_copy(
                ssts[b].at[pl.ds(0, PGROUP)],
                psrc_hbm.at[b, w, pl.ds(nf_b * 2 * PGROUP, PGROUP)])
            pltpu.sync_copy(
                dsts[b].at[pl.ds(0, PGROUP)],
                pdst_hbm.at[b, w, pl.ds(nf_b * 2 * PGROUP, PGROUP)])

        @pl.when(ngr >= 2)
        def _(b=b, nf_b=nf_b):
            pltpu.sync_copy(
                ssts[b].at[pl.ds(PGROUP, PGROUP)],
                psrc_hbm.at[b, w, pl.ds(nf_b * 2 * PGROUP + PGROUP, PGROUP)])
            pltpu.sync_copy(
                dsts[b].at[pl.ds(PGROUP, PGROUP)],
                pdst_hbm.at[b, w, pl.ds(nf_b * 2 * PGROUP + PGROUP, PGROUP)])

        total = nf_b * 2 + ngr
        cbuf[b, pl.ds(0, 16)] = jnp.full((16,), 1, jnp.int32) * total
        pltpu.sync_copy(cbuf.at[b], cnt_hbm.at[b * NW + w])


def _sc_partition(srcp, dstp):
    mesh = plsc.VectorSubcoreMesh(core_axis_name="c", subcore_axis_name="s")
    f = pl.kernel(
        _sc_partition_body,
        out_type=(
            jax.ShapeDtypeStruct((NBUCKET, NW, SLOTCAP), jnp.int32),
            jax.ShapeDtypeStruct((NBUCKET, NW, SLOTCAP), jnp.int32),
            jax.ShapeDtypeStruct((NBUCKET * NW, 16), jnp.int32),
        ),
        mesh=mesh,
        scratch_types=(
            [pltpu.VMEM((PGROUP,), jnp.int32)] * 2
            + [pltpu.VMEM((STAGECAP,), jnp.int32)] * NBUCKET
            + [pltpu.VMEM((STAGECAP,), jnp.int32)] * NBUCKET
            + [pltpu.VMEM((NBUCKET, 16), jnp.int32)]
        ),
        compiler_params=pltpu.CompilerParams(use_tc_tiling_on_sc=False,
                                             needs_layout_passes=False),
    )
    return f(srcp, dstp)


def _sc_scatter_body(m_hbm, psrc_hbm, pdst_hbm, cnt_hbm, out_hbm, *scr):
    srcgs = scr[0:2]
    dstgs = scr[2:4]
    idxs = scr[4:4 + NSLOT]
    rows = scr[4 + NSLOT:4 + 2 * NSLOT]
    zrow_v = scr[4 + 2 * NSLOT]
    cntv = scr[5 + 2 * NSLOT]
    acc = scr[6 + 2 * NSLOT]
    gsems = scr[7 + 2 * NSLOT:7 + 3 * NSLOT]
    ssems = scr[7 + 3 * NSLOT:7 + 4 * NSLOT]
    pfsems = scr[7 + 4 * NSLOT:9 + 4 * NSLOT]
    c = lax.axis_index("c")
    s = lax.axis_index("s")

    zero16 = jnp.zeros((16,), jnp.float32)

    def zrow_body(i, carry):
        for k in range(H // 16):
            zrow_v[i, pl.ds(k * 16, 16)] = zero16
        return carry

    lax.fori_loop(0, ZROWS, zrow_body, 0)

    for p in range(NPHASE):
        node_base = c * HALF + p * QUART
        bucket = c * NPHASE + p
        zrow0 = s * ZPS
        for j in range(ZPS // ZROWS):
            pltpu.sync_copy(zrow_v, acc.at[pl.ds(zrow0 + j * ZROWS, ZROWS)])
        rem = ZPS % ZROWS
        if rem:
            pltpu.sync_copy(zrow_v.at[pl.ds(0, rem)],
                            acc.at[pl.ds(zrow0 + (ZPS // ZROWS) * ZROWS, rem)])

        plsc.subcore_barrier()

        for sl in range(2):
            w = s * 2 + sl

            pltpu.sync_copy(cnt_hbm.at[bucket * NW + w], cntv)
            n = jnp.max(cntv[...])

            def fire_stage(o, u):
                pltpu.async_copy(
                    psrc_hbm.at[bucket, w, pl.ds(o * GROUP, GROUP)],
                    srcgs[u], pfsems[u])
                pltpu.async_copy(
                    pdst_hbm.at[bucket, w, pl.ds(o * GROUP, GROUP)],
                    dstgs[u], pfsems[u])

            def wait_stage(u):
                pltpu.make_async_copy(
                    psrc_hbm.at[0, 0, pl.ds(0, GROUP)], srcgs[u],
                    pfsems[u]).wait()
                pltpu.make_async_copy(
                    pdst_hbm.at[0, 0, pl.ds(0, GROUP)], dstgs[u],
                    pfsems[u]).wait()

            @pl.when(n > 0)
            def _():
                fire_stage(0, 0)

            @pl.when(n > 1)
            def _():
                fire_stage(1, 1)

            def process(o, u):
                wait_stage(u)
                gathers = []
                for b in range(NSLOT):
                    @pl.when(o > 0)
                    def _(b=b):
                        pltpu.make_async_copy(rows[b], acc.at[idxs[b]],
                                              ssems[b]).wait()
                    gathers.append(
                        pltpu.async_copy(
                            m_hbm.at[srcgs[u].at[pl.ds(b * CHUNK, CHUNK)]],
                            rows[b], gsems[b]))
                for b in range(NSLOT):
                    for k in range(CHUNK // 16):
                        d = dstgs[u][pl.ds(b * CHUNK + k * 16, 16)]
                        loc = d - node_base
                        oob = (loc < 0) | (loc >= QUART)
                        idxs[b][pl.ds(k * 16, 16)] = jnp.where(oob, QUART, loc)
                    gathers[b].wait()
                    pltpu.async_copy(rows[b], acc.at[idxs[b]], ssems[b],
                                     add=True)
                @pl.when(o + 2 < n)
                def _():
                    fire_stage(o + 2, u)

            def pair_body(t, carry):
                for u in range(2):
                    o = 2 * t + u

                    @pl.when(o < n)
                    def _(o=o, u=u):
                        process(o, u)
                return carry

            lax.fori_loop(0, (n + 1) // 2, pair_body, 0)

            @pl.when(n > 0)
            def _():
                for b in range(NSLOT):
                    pltpu.make_async_copy(rows[b], acc.at[idxs[b]],
                                          ssems[b]).wait()

        plsc.subcore_barrier()
        @pl.when(s < 12)
        def _(p=p):
            roff = s * 781
            pltpu.sync_copy(acc.at[pl.ds(roff, 781)],
                            out_hbm.at[c, pl.ds(p * QUART + roff, 781)])

        @pl.when(s >= 12)
        def _(p=p):
            roff = 12 * 781 + (s - 12) * 782
            pltpu.sync_copy(acc.at[pl.ds(roff, 782)],
                            out_hbm.at[c, pl.ds(p * QUART + roff, 782)])

        plsc.subcore_barrier()


def _sc_scatter(m, psrc, pdst, cnts):
    mesh = plsc.VectorSubcoreMesh(core_axis_name="c", subcore_axis_name="s")
    f = pl.kernel(
        _sc_scatter_body,
        out_type=jax.ShapeDtypeStruct((NCORES, ACC_ROWS, H), jnp.float32),
        mesh=mesh,
        scratch_types=(
            [pltpu.VMEM((GROUP,), jnp.int32)] * 4
            + [pltpu.VMEM((CHUNK,), jnp.int32)] * NSLOT
            + [pltpu.VMEM((CHUNK, H), jnp.float32)] * NSLOT
            + [pltpu.VMEM((ZROWS, H), jnp.float32)]
            + [pltpu.VMEM((16,), jnp.int32)]
            + [pltpu.VMEM_SHARED((ACC2_ROWS, H), jnp.float32)]
            + [pltpu.SemaphoreType.DMA] * (2 * NSLOT + 2)
        ),
        compiler_params=pltpu.CompilerParams(use_tc_tiling_on_sc=False,
                                             needs_layout_passes=False),
    )
    return f(m, psrc, pdst, cnts)


def _pre_body(x_ref, emb_ref, w1_ref, h0_ref, m1_ref):
    codes = x_ref[...]
    onehot = (codes == lax.broadcasted_iota(jnp.int32, (1, CODES), 1)
              ).astype(jnp.float32)
    h0 = lax.dot_general(onehot, emb_ref[...], (((1,), (0,)), ((), ())),
                         preferred_element_type=jnp.float32)
    h0_ref[...] = h0
    m1_ref[...] = jnp.dot(h0, w1_ref[...], preferred_element_type=jnp.float32)


def _pre(x, emb_pad, w1):
    return pl.pallas_call(
        _pre_body,
        grid=(GRID,),
        in_specs=[
            pl.BlockSpec((BN, 1), lambda i: (i, 0)),
            pl.BlockSpec((CODES, H), lambda i: (0, 0)),
            pl.BlockSpec((H, H), lambda i: (0, 0)),
        ],
        out_specs=[
            pl.BlockSpec((BN, H), lambda i: (i, 0)),
            pl.BlockSpec((BN, H), lambda i: (i, 0)),
        ],
        out_shape=[jax.ShapeDtypeStruct((N, H), jnp.float32)] * 2,
    )(x, emb_pad, w1)


def _gru_body(h_ref, a_ref, wihT, whhT, brz, bn_i, bn_h, wnext,
              hn_ref, mn_ref):
    h = h_ref[...]
    a = a_ref[0]
    gi = jnp.dot(a, wihT[...], preferred_element_type=jnp.float32)
    gh = jnp.dot(h, whhT[...], preferred_element_type=jnp.float32)
    rz = jax.nn.sigmoid(gi[:, :2 * H] + gh[:, :2 * H] + brz[...])
    r = rz[:, :H]
    z = rz[:, H:]
    n = jnp.tanh(gi[:, 2 * H:] + bn_i[...]
                 + r * (gh[:, 2 * H:] + bn_h[...]))
    hn = jax.nn.relu((1.0 - z) * n + z * h)
    hn_ref[...] = hn
    if mn_ref is not None:
        mn_ref[...] = jnp.dot(hn, wnext[...], preferred_element_type=jnp.float32)


def _gru(h, agg, wihT, whhT, brz, bn_i, bn_h, wnext):
    has_next = wnext is not None
    if not has_next:
        wnext = jnp.zeros((H, H), jnp.float32)
    body = (_gru_body if has_next
            else (lambda *refs: _gru_body(*refs, None)))
    per_core = HALF // BN
    out_shape = [jax.ShapeDtypeStruct((N, H), jnp.float32)]
    out_specs = [pl.BlockSpec((BN, H), lambda i: (i, 0))]
    if has_next:
        out_shape.append(jax.ShapeDtypeStruct((N, H), jnp.float32))
        out_specs.append(pl.BlockSpec((BN, H), lambda i: (i, 0)))
    return pl.pallas_call(
        body,
        grid=(GRID,),
        in_specs=[
            pl.BlockSpec((BN, H), lambda i: (i, 0)),
            pl.BlockSpec((1, BN, H), lambda i: (i // per_core, i % per_core, 0)),
            pl.BlockSpec((H, 3 * H), lambda i: (0, 0)),
            pl.BlockSpec((H, 3 * H), lambda i: (0, 0)),
            pl.BlockSpec((1, 2 * H), lambda i: (0, 0)),
            pl.BlockSpec((1, H), lambda i: (0, 0)),
            pl.BlockSpec((1, H), lambda i: (0, 0)),
            pl.BlockSpec((H, H), lambda i: (0, 0)),
        ],
        out_specs=out_specs,
        out_shape=out_shape,
    )(h, agg, wihT, whhT, brz, bn_i, bn_h, wnext)


BNP = 10000
GRIDP = N // BNP


def _pool_body(batch_ref, h_ref, f1w, f1b, f2w, f2b, f3w, f3b,
               out_ref, sums, cnt):
    i = pl.program_id(0)

    @pl.when(i == 0)
    def _():
        sums[...] = jnp.zeros_like(sums)
        cnt[...] = jnp.zeros_like(cnt)

    onehot = (batch_ref[...] == lax.broadcasted_iota(jnp.int32, (1, NG), 1)
              ).astype(jnp.float32)
    h = h_ref[...]
    sums[...] += lax.dot_general(onehot, h, (((0,), (0,)), ((), ())),
                                 preferred_element_type=jnp.float32)
    cnt[...] += lax.dot_general(onehot, jnp.ones((BNP, 1), jnp.float32),
                                (((0,), (0,)), ((), ())),
                                preferred_element_type=jnp.float32)

    @pl.when(i == GRIDP - 1)
    def _():
        pooled = sums[...] / jnp.maximum(cnt[...], 1.0)
        o = jax.nn.relu(jnp.dot(pooled, f1w[...],
                                preferred_element_type=jnp.float32) + f1b[...])
        o = jax.nn.relu(jnp.dot(o, f2w[...],
                                preferred_element_type=jnp.float32) + f2b[...])
        o = jnp.dot(o, f3w[...], preferred_element_type=jnp.float32) + f3b[...]
        out_ref[...] = o


def _pool(batch2d, h, f1w, f1b, f2w, f2b, f3w, f3b):
    return pl.pallas_call(
        _pool_body,
        grid=(GRIDP,),
        in_specs=[
            pl.BlockSpec((BNP, 1), lambda i: (i, 0)),
            pl.BlockSpec((BNP, H), lambda i: (i, 0)),
            pl.BlockSpec((H, H // 2), lambda i: (0, 0)),
            pl.BlockSpec((1, H // 2), lambda i: (0, 0)),
            pl.BlockSpec((H // 2, H // 4), lambda i: (0, 0)),
            pl.BlockSpec((1, H // 4), lambda i: (0, 0)),
            pl.BlockSpec((H // 4, 1), lambda i: (0, 0)),
            pl.BlockSpec((1, 1), lambda i: (0, 0)),
        ],
        out_specs=pl.BlockSpec((NG, 1), lambda i: (0, 0)),
        out_shape=jax.ShapeDtypeStruct((NG, 1), jnp.float32),
        scratch_shapes=[
            pltpu.VMEM((NG, NG), jnp.float32),
            pltpu.VMEM((NG, 1), jnp.float32),
        ],
    )(batch2d, h, f1w, f1b, f2w, f2b, f3w, f3b)


def kernel(x, edge_index, edge_attr, batch, node_emb, edge_lin_w, edge_lin_b,
           conv_weight, gru_Wih, gru_Whh, gru_bih, gru_bhh,
           fc1_w, fc1_b, fc2_w, fc2_b, fc3_w, fc3_b):
    src = edge_index[0]
    dst = edge_index[1]
    pad = EPAD - E
    srcp = jnp.concatenate([src, jnp.zeros((pad,), jnp.int32)])
    dstp = jnp.concatenate([dst, jnp.full((pad,), -1, jnp.int32)])

    emb_pad = jnp.pad(node_emb, ((0, CODES - node_emb.shape[0]), (0, 0)))

    h, m = _pre(x, emb_pad, conv_weight[0])
    psrc, pdst, cnts = _sc_partition(srcp, dstp)

    for i in range(NUM_LAYERS):
        agg = _sc_scatter(m, psrc, pdst, cnts)
        wihT = gru_Wih[i].T
        whhT = gru_Whh[i].T
        brz = (gru_bih[i, :2 * H] + gru_bhh[i, :2 * H]).reshape(1, 2 * H)
        bn_i = gru_bih[i, 2 * H:].reshape(1, H)
        bn_h = gru_bhh[i, 2 * H:].reshape(1, H)
        wnext = conv_weight[i + 1] if i + 1 < NUM_LAYERS else None
        res = _gru(h, agg, wihT, whhT, brz, bn_i, bn_h, wnext)
        if wnext is not None:
            h, m = res
        else:
            h = res[0]

    out = _pool(batch.reshape(N, 1), h,
                fc1_w.T, fc1_b.reshape(1, H // 2),
                fc2_w.T, fc2_b.reshape(1, H // 4),
                fc3_w.T, fc3_b.reshape(1, 1))
    return out[:, 0]


# speedup vs baseline: 1.0019x; 1.0019x over previous
import jax
import jax.numpy as jnp
from jax import lax
from jax.experimental import pallas as pl
from jax.experimental.pallas import tpu as pltpu
from jax.experimental.pallas import tpu_sc as plsc

N = 50000
E = 800000
H = 64
NG = 64
NUM_LAYERS = 3

NCORES = 2
NSUB = 16
HALF = N // NCORES
CHUNK = 512
CROWS = CHUNK // 128
CHUNKS_PER_SUB = 100
EPAD = NSUB * CHUNKS_PER_SUB * CHUNK
ACC_ROWS = 25088
NW = 32
EPW = EPAD // NW
NBUCKET = 4
PGROUP = 1024
SLOTCAP = EPW + PGROUP
STAGECAP = 2 * PGROUP + PGROUP + 16
TRASH_OFF = 3 * PGROUP
NPHASE = 2
QUART = HALF // NPHASE
ACC2_ROWS = 12512
ZROWS = 128
ZPS = ACC2_ROWS // NSUB

BN = 5000
GRID = N // BN
CODES = 104


NSLOT = 2
GROUP = NSLOT * CHUNK
GROUPS_PER_SUB = CHUNKS_PER_SUB // NSLOT
ROWBYTES = CHUNK * H * 4


def _sc_partition_body(src_hbm, dst_hbm, psrc_hbm, pdst_hbm, cnt_hbm, *scr):
    sg, dg = scr[0], scr[1]
    ssts = scr[2:2 + NBUCKET]
    dsts = scr[2 + NBUCKET:2 + 2 * NBUCKET]
    cbuf = scr[2 + 2 * NBUCKET]
    c = lax.axis_index("c")
    s = lax.axis_index("s")
    w = c * NSUB + s
    ebase = w * EPW

    def group_body(g, carry):
        pltpu.sync_copy(src_hbm.at[pl.ds(ebase + g * PGROUP, PGROUP)], sg)
        pltpu.sync_copy(dst_hbm.at[pl.ds(ebase + g * PGROUP, PGROUP)], dg)

        def sub_body(i, inner):
            d = dg[pl.ds(i * 16, 16)]
            v = sg[pl.ds(i * 16, 16)]
            out = []
            lane = lax.iota(jnp.int32, 16)
            for b in range(NBUCKET):
                cnt_b, nf_b = inner[2 * b], inner[2 * b + 1]
                mb = (d >= b * QUART) & (d < (b + 1) * QUART)
                ps = plsc.cumsum(mb.astype(jnp.int32))
                tgt = jnp.where(mb, cnt_b + ps - 1, TRASH_OFF + lane)
                plsc.store_scatter(ssts[b], [tgt], v)
                plsc.store_scatter(dsts[b], [tgt], d)
                cnt_b = cnt_b + ps[15]

                def flush(cn, nf, b=b):
                    pltpu.sync_copy(
                        ssts[b].at[pl.ds(0, 2 * PGROUP)],
                        psrc_hbm.at[b, w, pl.ds(nf * 2 * PGROUP, 2 * PGROUP)])
                    pltpu.sync_copy(
                        dsts[b].at[pl.ds(0, 2 * PGROUP)],
                        pdst_hbm.at[b, w, pl.ds(nf * 2 * PGROUP, 2 * PGROUP)])
                    ssts[b][pl.ds(0, 16)] = ssts[b][pl.ds(2 * PGROUP, 16)]
                    dsts[b][pl.ds(0, 16)] = dsts[b][pl.ds(2 * PGROUP, 16)]
                    return cn - 2 * PGROUP, nf + 1

                cnt_b, nf_b = lax.cond(cnt_b >= 2 * PGROUP, flush,
                                       lambda cn, nf: (cn, nf), cnt_b, nf_b)
                out += [cnt_b, nf_b]
            return tuple(out)

        return lax.fori_loop(0, PGROUP // 16, sub_body, carry)

    state = lax.fori_loop(0, EPW // PGROUP, group_body,
                          tuple([jnp.int32(0)] * (2 * NBUCKET)))

    trash_src = jnp.zeros((16,), jnp.int32)
    trash_dst = jnp.full((16,), -1, jnp.int32)
    for b in range(NBUCKET):
        cnt_b, nf_b = state[2 * b], state[2 * b + 1]
        for t in range(PGROUP // 16):
            ssts[b][pl.ds(cnt_b + t * 16, 16)] = trash_src
            dsts[b][pl.ds(cnt_b + t * 16, 16)] = trash_dst
        ngr = (cnt_b + PGROUP - 1) // PGROUP

        @pl.when(ngr >= 1)
        def _(b=b, nf_b=nf_b):
            pltpu.sync_copy(
                ssts[b].at[pl.ds(0, PGROUP)],
                psrc_hbm.at[b, w, pl.ds(nf_b * 2 * PGROUP, PGROUP)])
            pltpu.sync_copy(
                dsts[b].at[pl.ds(0, PGROUP)],
                pdst_hbm.at[b, w, pl.ds(nf_b * 2 * PGROUP, PGROUP)])

        @pl.when(ngr >= 2)
        def _(b=b, nf_b=nf_b):
            pltpu.sync_copy(
                ssts[b].at[pl.ds(PGROUP, PGROUP)],
                psrc_hbm.at[b, w, pl.ds(nf_b * 2 * PGROUP + PGROUP, PGROUP)])
            pltpu.sync_copy(
                dsts[b].at[pl.ds(PGROUP, PGROUP)],
                pdst_hbm.at[b, w, pl.ds(nf_b * 2 * PGROUP + PGROUP, PGROUP)])

        total = nf_b * 2 + ngr
        cbuf[b, pl.ds(0, 16)] = jnp.full((16,), 1, jnp.int32) * total
        pltpu.sync_copy(cbuf.at[b], cnt_hbm.at[b * NW + w])


def _sc_partition(srcp, dstp):
    mesh = plsc.VectorSubcoreMesh(core_axis_name="c", subcore_axis_name="s")
    f = pl.kernel(
        _sc_partition_body,
        out_type=(
            jax.ShapeDtypeStruct((NBUCKET, NW, SLOTCAP), jnp.int32),
            jax.ShapeDtypeStruct((NBUCKET, NW, SLOTCAP), jnp.int32),
            jax.ShapeDtypeStruct((NBUCKET * NW, 16), jnp.int32),
        ),
        mesh=mesh,
        scratch_types=(
            [pltpu.VMEM((PGROUP,), jnp.int32)] * 2
            + [pltpu.VMEM((STAGECAP,), jnp.int32)] * NBUCKET
            + [pltpu.VMEM((STAGECAP,), jnp.int32)] * NBUCKET
            + [pltpu.VMEM((NBUCKET, 16), jnp.int32)]
        ),
        compiler_params=pltpu.CompilerParams(use_tc_tiling_on_sc=False,
                                             needs_layout_passes=False),
    )
    return f(srcp, dstp)


def _sc_scatter_body(m_hbm, psrc_hbm, pdst_hbm, cnt_hbm, out_hbm, *scr):
    srcgs = scr[0:2]
    dstgs = scr[2:4]
    idxs = scr[4:4 + NSLOT]
    rows = scr[4 + NSLOT:4 + 2 * NSLOT]
    zrow_v = scr[4 + 2 * NSLOT]
    cntv = scr[5 + 2 * NSLOT]
    acc = scr[6 + 2 * NSLOT]
    gsems = scr[7 + 2 * NSLOT:7 + 3 * NSLOT]
    ssems = scr[7 + 3 * NSLOT:7 + 4 * NSLOT]
    pfsems = scr[7 + 4 * NSLOT:9 + 4 * NSLOT]
    c = lax.axis_index("c")
    s = lax.axis_index("s")

    zero16 = jnp.zeros((16,), jnp.float32)

    def zrow_body(i, carry):
        for k in range(H // 16):
            zrow_v[i, pl.ds(k * 16, 16)] = zero16
        return carry

    lax.fori_loop(0, ZROWS, zrow_body, 0)

    for p in range(NPHASE):
        node_base = c * HALF + p * QUART
        bucket = c * NPHASE + p
        zrow0 = s * ZPS
        for j in range(ZPS // ZROWS):
            pltpu.sync_copy(zrow_v, acc.at[pl.ds(zrow0 + j * ZROWS, ZROWS)])
        rem = ZPS % ZROWS
        if rem:
            pltpu.sync_copy(zrow_v.at[pl.ds(0, rem)],
                            acc.at[pl.ds(zrow0 + (ZPS // ZROWS) * ZROWS, rem)])

        plsc.subcore_barrier()

        for sl in range(2):
            w = s * 2 + sl

            pltpu.sync_copy(cnt_hbm.at[bucket * NW + w], cntv)
            n = jnp.max(cntv[...])

            def fire_stage(o, u):
                pltpu.async_copy(
                    psrc_hbm.at[bucket, w, pl.ds(o * GROUP, GROUP)],
                    srcgs[u], pfsems[u])
                pltpu.async_copy(
                    pdst_hbm.at[bucket, w, pl.ds(o * GROUP, GROUP)],
                    dstgs[u], pfsems[u])

            def wait_stage(u):
                pltpu.make_async_copy(
                    psrc_hbm.at[0, 0, pl.ds(0, GROUP)], srcgs[u],
                    pfsems[u]).wait()
                pltpu.make_async_copy(
                    pdst_hbm.at[0, 0, pl.ds(0, GROUP)], dstgs[u],
                    pfsems[u]).wait()

            @pl.when(n > 0)
            def _():
                fire_stage(0, 0)

            @pl.when(n > 1)
            def _():
                fire_stage(1, 1)

            def process(o, u):
                wait_stage(u)
                gathers = []
                for b in range(NSLOT):
                    @pl.when(o > 0)
                    def _(b=b):
                        pltpu.make_async_copy(rows[b], acc.at[idxs[b]],
                                              ssems[b]).wait()
                    gathers.append(
                        pltpu.async_copy(
                            m_hbm.at[srcgs[u].at[pl.ds(b * CHUNK, CHUNK)]],
                            rows[b], gsems[b]))
                for b in range(NSLOT):
                    for k in range(CHUNK // 16):
                        d = dstgs[u][pl.ds(b * CHUNK + k * 16, 16)]
                        loc = d - node_base
                        oob = (loc < 0) | (loc >= QUART)
                        idxs[b][pl.ds(k * 16, 16)] = jnp.where(oob, QUART, loc)
                    gathers[b].wait()
                    pltpu.async_copy(rows[b], acc.at[idxs[b]], ssems[b],
                                     add=True)
                @pl.when(o + 2 < n)
                def _():
                    fire_stage(o + 2, u)

            def pair_body(t, carry):
                for u in range(2):
                    process(2 * t + u, u)
                return carry

            lax.fori_loop(0, n // 2, pair_body, 0)

            @pl.when(n % 2 == 1)
            def _():
                process(n - 1, 0)

            @pl.when(n > 0)
            def _():
                for b in range(NSLOT):
                    pltpu.make_async_copy(rows[b], acc.at[idxs[b]],
                                          ssems[b]).wait()

        plsc.subcore_barrier()
        @pl.when(s < 12)
        def _(p=p):
            roff = s * 781
            pltpu.sync_copy(acc.at[pl.ds(roff, 781)],
                            out_hbm.at[c, pl.ds(p * QUART + roff, 781)])

        @pl.when(s >= 12)
        def _(p=p):
            roff = 12 * 781 + (s - 12) * 782
            pltpu.sync_copy(acc.at[pl.ds(roff, 782)],
                            out_hbm.at[c, pl.ds(p * QUART + roff, 782)])

        plsc.subcore_barrier()


def _sc_scatter(m, psrc, pdst, cnts):
    mesh = plsc.VectorSubcoreMesh(core_axis_name="c", subcore_axis_name="s")
    f = pl.kernel(
        _sc_scatter_body,
        out_type=jax.ShapeDtypeStruct((NCORES, ACC_ROWS, H), jnp.float32),
        mesh=mesh,
        scratch_types=(
            [pltpu.VMEM((GROUP,), jnp.int32)] * 4
            + [pltpu.VMEM((CHUNK,), jnp.int32)] * NSLOT
            + [pltpu.VMEM((CHUNK, H), jnp.float32)] * NSLOT
            + [pltpu.VMEM((ZROWS, H), jnp.float32)]
            + [pltpu.VMEM((16,), jnp.int32)]
            + [pltpu.VMEM_SHARED((ACC2_ROWS, H), jnp.float32)]
            + [pltpu.SemaphoreType.DMA] * (2 * NSLOT + 2)
        ),
        compiler_params=pltpu.CompilerParams(use_tc_tiling_on_sc=False,
                                             needs_layout_passes=False),
    )
    return f(m, psrc, pdst, cnts)


def _pre_body(x_ref, emb_ref, w1_ref, h0_ref, m1_ref):
    codes = x_ref[...]
    onehot = (codes == lax.broadcasted_iota(jnp.int32, (1, CODES), 1)
              ).astype(jnp.float32)
    h0 = lax.dot_general(onehot, emb_ref[...], (((1,), (0,)), ((), ())),
                         preferred_element_type=jnp.float32)
    h0_ref[...] = h0
    m1_ref[...] = jnp.dot(h0, w1_ref[...], preferred_element_type=jnp.float32)


def _pre(x, emb_pad, w1):
    return pl.pallas_call(
        _pre_body,
        grid=(GRID,),
        in_specs=[
            pl.BlockSpec((BN, 1), lambda i: (i, 0)),
            pl.BlockSpec((CODES, H), lambda i: (0, 0)),
            pl.BlockSpec((H, H), lambda i: (0, 0)),
        ],
        out_specs=[
            pl.BlockSpec((BN, H), lambda i: (i, 0)),
            pl.BlockSpec((BN, H), lambda i: (i, 0)),
        ],
        out_shape=[jax.ShapeDtypeStruct((N, H), jnp.float32)] * 2,
    )(x, emb_pad, w1)


def _gru_body(h_ref, a_ref, wihT, whhT, brz, bn_i, bn_h, wnext,
              hn_ref, mn_ref):
    h = h_ref[...]
    a = a_ref[0]
    gi = jnp.dot(a, wihT[...], preferred_element_type=jnp.float32)
    gh = jnp.dot(h, whhT[...], preferred_element_type=jnp.float32)
    rz = jax.nn.sigmoid(gi[:, :2 * H] + gh[:, :2 * H] + brz[...])
    r = rz[:, :H]
    z = rz[:, H:]
    n = jnp.tanh(gi[:, 2 * H:] + bn_i[...]
                 + r * (gh[:, 2 * H:] + bn_h[...]))
    hn = jax.nn.relu((1.0 - z) * n + z * h)
    hn_ref[...] = hn
    mn_ref[...] = jnp.dot(hn, wnext[...], preferred_element_type=jnp.float32)


def _gru(h, agg, wihT, whhT, brz, bn_i, bn_h, wnext):
    per_core = HALF // BN
    return pl.pallas_call(
        _gru_body,
        grid=(GRID,),
        in_specs=[
            pl.BlockSpec((BN, H), lambda i: (i, 0)),
            pl.BlockSpec((1, BN, H), lambda i: (i // per_core, i % per_core, 0)),
            pl.BlockSpec((H, 3 * H), lambda i: (0, 0)),
            pl.BlockSpec((H, 3 * H), lambda i: (0, 0)),
            pl.BlockSpec((1, 2 * H), lambda i: (0, 0)),
            pl.BlockSpec((1, H), lambda i: (0, 0)),
            pl.BlockSpec((1, H), lambda i: (0, 0)),
            pl.BlockSpec((H, H), lambda i: (0, 0)),
        ],
        out_specs=[pl.BlockSpec((BN, H), lambda i: (i, 0))] * 2,
        out_shape=[jax.ShapeDtypeStruct((N, H), jnp.float32)] * 2,
    )(h, agg, wihT, whhT, brz, bn_i, bn_h, wnext)


def _gru_pool_body(h_ref, a_ref, wihT, whhT, brz, bn_i, bn_h,
                   batch_ref, f1w, f1b, f2w, f2b, f3w, f3b,
                   out_ref, sums, cnt):
    i = pl.program_id(0)

    @pl.when(i == 0)
    def _():
        sums[...] = jnp.zeros_like(sums)
        cnt[...] = jnp.zeros_like(cnt)

    h = h_ref[...]
    a = a_ref[0]
    gi = jnp.dot(a, wihT[...], preferred_element_type=jnp.float32)
    gh = jnp.dot(h, whhT[...], preferred_element_type=jnp.float32)
    rz = jax.nn.sigmoid(gi[:, :2 * H] + gh[:, :2 * H] + brz[...])
    r = rz[:, :H]
    z = rz[:, H:]
    n = jnp.tanh(gi[:, 2 * H:] + bn_i[...]
                 + r * (gh[:, 2 * H:] + bn_h[...]))
    hn = jax.nn.relu((1.0 - z) * n + z * h)

    onehot = (batch_ref[...] == lax.broadcasted_iota(jnp.int32, (1, NG), 1)
              ).astype(jnp.float32)
    sums[...] += lax.dot_general(onehot, hn, (((0,), (0,)), ((), ())),
                                 preferred_element_type=jnp.float32)
    cnt[...] += lax.dot_general(onehot, jnp.ones((BN, 1), jnp.float32),
                                (((0,), (0,)), ((), ())),
                                preferred_element_type=jnp.float32)

    @pl.when(i == GRID - 1)
    def _():
        pooled = sums[...] / jnp.maximum(cnt[...], 1.0)
        o = jax.nn.relu(jnp.dot(pooled, f1w[...],
                                preferred_element_type=jnp.float32) + f1b[...])
        o = jax.nn.relu(jnp.dot(o, f2w[...],
                                preferred_element_type=jnp.float32) + f2b[...])
        o = jnp.dot(o, f3w[...], preferred_element_type=jnp.float32) + f3b[...]
        out_ref[...] = o


def _gru_pool(h, agg, wihT, whhT, brz, bn_i, bn_h, batch2d,
              f1w, f1b, f2w, f2b, f3w, f3b):
    per_core = HALF // BN
    return pl.pallas_call(
        _gru_pool_body,
        grid=(GRID,),
        in_specs=[
            pl.BlockSpec((BN, H), lambda i: (i, 0)),
            pl.BlockSpec((1, BN, H), lambda i: (i // per_core, i % per_core, 0)),
            pl.BlockSpec((H, 3 * H), lambda i: (0, 0)),
            pl.BlockSpec((H, 3 * H), lambda i: (0, 0)),
            pl.BlockSpec((1, 2 * H), lambda i: (0, 0)),
            pl.BlockSpec((1, H), lambda i: (0, 0)),
            pl.BlockSpec((1, H), lambda i: (0, 0)),
            pl.BlockSpec((BN, 1), lambda i: (i, 0)),
            pl.BlockSpec((H, H // 2), lambda i: (0, 0)),
            pl.BlockSpec((1, H // 2), lambda i: (0, 0)),
            pl.BlockSpec((H // 2, H // 4), lambda i: (0, 0)),
            pl.BlockSpec((1, H // 4), lambda i: (0, 0)),
            pl.BlockSpec((H // 4, 1), lambda i: (0, 0)),
            pl.BlockSpec((1, 1), lambda i: (0, 0)),
        ],
        out_specs=pl.BlockSpec((NG, 1), lambda i: (0, 0)),
        out_shape=jax.ShapeDtypeStruct((NG, 1), jnp.float32),
        scratch_shapes=[
            pltpu.VMEM((NG, NG), jnp.float32),
            pltpu.VMEM((NG, 1), jnp.float32),
        ],
    )(h, agg, wihT, whhT, brz, bn_i, bn_h, batch2d,
      f1w, f1b, f2w, f2b, f3w, f3b)


def kernel(x, edge_index, edge_attr, batch, node_emb, edge_lin_w, edge_lin_b,
           conv_weight, gru_Wih, gru_Whh, gru_bih, gru_bhh,
           fc1_w, fc1_b, fc2_w, fc2_b, fc3_w, fc3_b):
    src = edge_index[0]
    dst = edge_index[1]
    pad = EPAD - E
    srcp = jnp.concatenate([src, jnp.zeros((pad,), jnp.int32)])
    dstp = jnp.concatenate([dst, jnp.full((pad,), -1, jnp.int32)])

    emb_pad = jnp.pad(node_emb, ((0, CODES - node_emb.shape[0]), (0, 0)))

    h, m = _pre(x, emb_pad, conv_weight[0])
    psrc, pdst, cnts = _sc_partition(srcp, dstp)

    for i in range(NUM_LAYERS):
        agg = _sc_scatter(m, psrc, pdst, cnts)
        wihT = gru_Wih[i].T
        whhT = gru_Whh[i].T
        brz = (gru_bih[i, :2 * H] + gru_bhh[i, :2 * H]).reshape(1, 2 * H)
        bn_i = gru_bih[i, 2 * H:].reshape(1, H)
        bn_h = gru_bhh[i, 2 * H:].reshape(1, H)
        if i + 1 < NUM_LAYERS:
            h, m = _gru(h, agg, wihT, whhT, brz, bn_i, bn_h,
                        conv_weight[i + 1])
        else:
            out = _gru_pool(h, agg, wihT, whhT, brz, bn_i, bn_h,
                            batch.reshape(N, 1),
                            fc1_w.T, fc1_b.reshape(1, H // 2),
                            fc2_w.T, fc2_b.reshape(1, H // 4),
                            fc3_w.T, fc3_b.reshape(1, 1))
    return out[:, 0]


# speedup vs baseline: 1.0052x; 1.0033x over previous
import jax
import jax.numpy as jnp
from jax import lax
from jax.experimental import pallas as pl
from jax.experimental.pallas import tpu as pltpu
from jax.experimental.pallas import tpu_sc as plsc

N = 50000
E = 800000
H = 64
NG = 64
NUM_LAYERS = 3

NCORES = 2
NSUB = 16
HALF = N // NCORES
CHUNK = 128
CHUNKS_PER_SUB = 400
EPAD = NSUB * CHUNKS_PER_SUB * CHUNK
ACC_ROWS = 25088
NW = 32
EPW = EPAD // NW
NBUCKET = 4
PGROUP = 1024
SLOTCAP = EPW + PGROUP
STAGECAP = 2 * PGROUP + PGROUP + 16
TRASH_OFF = 3 * PGROUP
NPHASE = 2
QUART = HALF // NPHASE
ACC2_ROWS = 25104
ZROWS = 128
ZPS = ACC2_ROWS // NSUB

BN = 5000
GRID = N // BN
CODES = 104


NSLOT = 2
GROUP = NSLOT * CHUNK
GROUPS_PER_SUB = CHUNKS_PER_SUB // NSLOT
ROWBYTES = CHUNK * H * 4


def _sc_partition_body(src_hbm, dst_hbm, psrc_hbm, pdst_hbm, cnt_hbm, *scr):
    sg, dg = scr[0], scr[1]
    ssts = scr[2:2 + NBUCKET]
    dsts = scr[2 + NBUCKET:2 + 2 * NBUCKET]
    cbuf = scr[2 + 2 * NBUCKET]
    c = lax.axis_index("c")
    s = lax.axis_index("s")
    w = c * NSUB + s
    ebase = w * EPW

    def group_body(g, carry):
        pltpu.sync_copy(src_hbm.at[pl.ds(ebase + g * PGROUP, PGROUP)], sg)
        pltpu.sync_copy(dst_hbm.at[pl.ds(ebase + g * PGROUP, PGROUP)], dg)

        def sub_body(i, inner):
            d = dg[pl.ds(i * 16, 16)]
            v = sg[pl.ds(i * 16, 16)]
            out = []
            lane = lax.iota(jnp.int32, 16)
            for b in range(NBUCKET):
                cnt_b, nf_b = inner[2 * b], inner[2 * b + 1]
                mb = (d >= b * QUART) & (d < (b + 1) * QUART)
                ps = plsc.cumsum(mb.astype(jnp.int32))
                tgt = jnp.where(mb, cnt_b + ps - 1, TRASH_OFF + lane)
                plsc.store_scatter(ssts[b], [tgt], v)
                plsc.store_scatter(dsts[b], [tgt], d)
                cnt_b = cnt_b + ps[15]

                def flush(cn, nf, b=b):
                    pltpu.sync_copy(
                        ssts[b].at[pl.ds(0, 2 * PGROUP)],
                        psrc_hbm.at[b, w, pl.ds(nf * 2 * PGROUP, 2 * PGROUP)])
                    pltpu.sync_copy(
                        dsts[b].at[pl.ds(0, 2 * PGROUP)],
                        pdst_hbm.at[b, w, pl.ds(nf * 2 * PGROUP, 2 * PGROUP)])
                    ssts[b][pl.ds(0, 16)] = ssts[b][pl.ds(2 * PGROUP, 16)]
                    dsts[b][pl.ds(0, 16)] = dsts[b][pl.ds(2 * PGROUP, 16)]
                    return cn - 2 * PGROUP, nf + 1

                cnt_b, nf_b = lax.cond(cnt_b >= 2 * PGROUP, flush,
                                       lambda cn, nf: (cn, nf), cnt_b, nf_b)
                out += [cnt_b, nf_b]
            return tuple(out)

        return lax.fori_loop(0, PGROUP // 16, sub_body, carry)

    state = lax.fori_loop(0, EPW // PGROUP, group_body,
                          tuple([jnp.int32(0)] * (2 * NBUCKET)))

    trash_src = jnp.zeros((16,), jnp.int32)
    trash_dst = jnp.full((16,), -1, jnp.int32)
    for b in range(NBUCKET):
        cnt_b, nf_b = state[2 * b], state[2 * b + 1]
        for t in range(PGROUP // 16):
            ssts[b][pl.ds(cnt_b + t * 16, 16)] = trash_src
            dsts[b][pl.ds(cnt_b + t * 16, 16)] = trash_dst
        ngr = (cnt_b + PGROUP - 1) // PGROUP

        @pl.when(ngr >= 1)
        def _(b=b, nf_b=nf_b):
            pltpu.sync_copy(
                ssts[b].at[pl.ds(0, PGROUP)],
                psrc_hbm.at[b, w, pl.ds(nf_b * 2 * PGROUP, PGROUP)])
            pltpu.sync_copy(
                dsts[b].at[pl.ds(0, PGROUP)],
                pdst_hbm.at[b, w, pl.ds(nf_b * 2 * PGROUP, PGROUP)])

        @pl.when(ngr >= 2)
        def _(b=b, nf_b=nf_b):
            pltpu.sync_copy(
                ssts[b].at[pl.ds(PGROUP, PGROUP)],
                psrc_hbm.at[b, w, pl.ds(nf_b * 2 * PGROUP + PGROUP, PGROUP)])
            pltpu.sync_copy(
                dsts[b].at[pl.ds(PGROUP, PGROUP)],
                pdst_hbm.at[b, w, pl.ds(nf_b * 2 * PGROUP + PGROUP, PGROUP)])

        total = nf_b * 2 + ngr
        cbuf[b, pl.ds(0, 16)] = jnp.full((16,), 1, jnp.int32) * total
        pltpu.sync_copy(cbuf.at[b], cnt_hbm.at[b * NW + w])


def _sc_partition(srcp, dstp):
    mesh = plsc.VectorSubcoreMesh(core_axis_name="c", subcore_axis_name="s")
    f = pl.kernel(
        _sc_partition_body,
        out_type=(
            jax.ShapeDtypeStruct((NBUCKET, NW, SLOTCAP), jnp.int32),
            jax.ShapeDtypeStruct((NBUCKET, NW, SLOTCAP), jnp.int32),
            jax.ShapeDtypeStruct((NBUCKET * NW, 16), jnp.int32),
        ),
        mesh=mesh,
        scratch_types=(
            [pltpu.VMEM((PGROUP,), jnp.int32)] * 2
            + [pltpu.VMEM((STAGECAP,), jnp.int32)] * NBUCKET
            + [pltpu.VMEM((STAGECAP,), jnp.int32)] * NBUCKET
            + [pltpu.VMEM((NBUCKET, 16), jnp.int32)]
        ),
        compiler_params=pltpu.CompilerParams(use_tc_tiling_on_sc=False,
                                             needs_layout_passes=False),
    )
    return f(srcp, dstp)


def _sc_scatter_body(m_hbm, psrc_hbm, pdst_hbm, cnt_hbm, out_hbm, *scr):
    srcgs = scr[0:2]
    dstgs = scr[2:4]
    idxs = scr[4:4 + NSLOT]
    rows = scr[4 + NSLOT:4 + 2 * NSLOT]
    zrow_v = scr[4 + 2 * NSLOT]
    cntv = scr[5 + 2 * NSLOT]
    acc = scr[6 + 2 * NSLOT]
    gsems = scr[7 + 2 * NSLOT:7 + 3 * NSLOT]
    ssems = scr[7 + 3 * NSLOT:7 + 4 * NSLOT]
    pfsems = scr[7 + 4 * NSLOT:9 + 4 * NSLOT]
    c = lax.axis_index("c")
    s = lax.axis_index("s")

    zero16 = jnp.zeros((16,), jnp.float32)

    def zrow_body(i, carry):
        for k in range(H // 16):
            zrow_v[i, pl.ds(k * 16, 16)] = zero16
        return carry

    lax.fori_loop(0, ZROWS, zrow_body, 0)

    node_base = c * HALF
    zrow0 = s * ZPS
    for j in range(ZPS // ZROWS):
        pltpu.sync_copy(zrow_v, acc.at[pl.ds(zrow0 + j * ZROWS, ZROWS)])
    rem = ZPS % ZROWS
    if rem:
        pltpu.sync_copy(zrow_v.at[pl.ds(0, rem)],
                        acc.at[pl.ds(zrow0 + (ZPS // ZROWS) * ZROWS, rem)])

    plsc.subcore_barrier()

    for bk in range(NPHASE):
        bucket = c * NPHASE + bk
        for sl in range(2):
            w = s * 2 + sl

            pltpu.sync_copy(cnt_hbm.at[bucket * NW + w], cntv)
            n = jnp.max(cntv[...]) * (PGROUP // GROUP)

            def fire_stage(o, u):
                pltpu.async_copy(
                    psrc_hbm.at[bucket, w, pl.ds(o * GROUP, GROUP)],
                    srcgs[u], pfsems[u])
                pltpu.async_copy(
                    pdst_hbm.at[bucket, w, pl.ds(o * GROUP, GROUP)],
                    dstgs[u], pfsems[u])

            def wait_stage(u):
                pltpu.make_async_copy(
                    psrc_hbm.at[0, 0, pl.ds(0, GROUP)], srcgs[u],
                    pfsems[u]).wait()
                pltpu.make_async_copy(
                    pdst_hbm.at[0, 0, pl.ds(0, GROUP)], dstgs[u],
                    pfsems[u]).wait()

            @pl.when(n > 0)
            def _():
                fire_stage(0, 0)

            @pl.when(n > 1)
            def _():
                fire_stage(1, 1)

            def process(o, u):
                wait_stage(u)
                gathers = []
                for b in range(NSLOT):
                    @pl.when(o > 0)
                    def _(b=b):
                        pltpu.make_async_copy(rows[b], acc.at[idxs[b]],
                                              ssems[b]).wait()
                    gathers.append(
                        pltpu.async_copy(
                            m_hbm.at[srcgs[u].at[pl.ds(b * CHUNK, CHUNK)]],
                            rows[b], gsems[b]))
                for b in range(NSLOT):
                    for k in range(CHUNK // 16):
                        d = dstgs[u][pl.ds(b * CHUNK + k * 16, 16)]
                        loc = d - node_base
                        oob = (loc < 0) | (loc >= HALF)
                        idxs[b][pl.ds(k * 16, 16)] = jnp.where(oob, HALF, loc)
                    gathers[b].wait()
                    pltpu.async_copy(rows[b], acc.at[idxs[b]], ssems[b],
                                     add=True)
                @pl.when(o + 2 < n)
                def _(o=o, u=u):
                    fire_stage(o + 2, u)

            def pair_body(t, carry):
                for u in range(2):
                    process(2 * t + u, u)
                return carry

            lax.fori_loop(0, n // 2, pair_body, 0)

            @pl.when(n % 2 == 1)
            def _():
                process(n - 1, 0)

            @pl.when(n > 0)
            def _():
                for b in range(NSLOT):
                    pltpu.make_async_copy(rows[b], acc.at[idxs[b]],
                                          ssems[b]).wait()

    plsc.subcore_barrier()
    @pl.when(s < 8)
    def _():
        roff = s * 1563
        pltpu.sync_copy(acc.at[pl.ds(roff, 1563)],
                        out_hbm.at[c, pl.ds(roff, 1563)])

    @pl.when(s >= 8)
    def _():
        roff = 8 * 1563 + (s - 8) * 1562
        pltpu.sync_copy(acc.at[pl.ds(roff, 1562)],
                        out_hbm.at[c, pl.ds(roff, 1562)])


def _sc_scatter(m, psrc, pdst, cnts):
    mesh = plsc.VectorSubcoreMesh(core_axis_name="c", subcore_axis_name="s")
    f = pl.kernel(
        _sc_scatter_body,
        out_type=jax.ShapeDtypeStruct((NCORES, ACC_ROWS, H), jnp.float32),
        mesh=mesh,
        scratch_types=(
            [pltpu.VMEM((GROUP,), jnp.int32)] * 4
            + [pltpu.VMEM((CHUNK,), jnp.int32)] * NSLOT
            + [pltpu.VMEM((CHUNK, H), jnp.float32)] * NSLOT
            + [pltpu.VMEM((ZROWS, H), jnp.float32)]
            + [pltpu.VMEM((16,), jnp.int32)]
            + [pltpu.VMEM_SHARED((ACC2_ROWS, H), jnp.float32)]
            + [pltpu.SemaphoreType.DMA] * (2 * NSLOT + 2)
        ),
        compiler_params=pltpu.CompilerParams(use_tc_tiling_on_sc=False,
                                             needs_layout_passes=False),
    )
    return f(m, psrc, pdst, cnts)


def _pre_body(x_ref, emb_ref, w1_ref, h0_ref, m1_ref):
    codes = x_ref[...]
    onehot = (codes == lax.broadcasted_iota(jnp.int32, (1, CODES), 1)
              ).astype(jnp.float32)
    h0 = lax.dot_general(onehot, emb_ref[...], (((1,), (0,)), ((), ())),
                         preferred_element_type=jnp.float32)
    h0_ref[...] = h0
    m1_ref[...] = jnp.dot(h0, w1_ref[...], preferred_element_type=jnp.float32)


def _pre(x, emb_pad, w1):
    return pl.pallas_call(
        _pre_body,
        grid=(GRID,),
        in_specs=[
            pl.BlockSpec((BN, 1), lambda i: (i, 0)),
            pl.BlockSpec((CODES, H), lambda i: (0, 0)),
            pl.BlockSpec((H, H), lambda i: (0, 0)),
        ],
        out_specs=[
            pl.BlockSpec((BN, H), lambda i: (i, 0)),
            pl.BlockSpec((BN, H), lambda i: (i, 0)),
        ],
        out_shape=[jax.ShapeDtypeStruct((N, H), jnp.float32)] * 2,
    )(x, emb_pad, w1)


def _gru_body(h_ref, a_ref, wihT, whhT, brz, bn_i, bn_h, wnext,
              hn_ref, mn_ref):
    h = h_ref[...]
    a = a_ref[0]
    gi = jnp.dot(a, wihT[...], preferred_element_type=jnp.float32)
    gh = jnp.dot(h, whhT[...], preferred_element_type=jnp.float32)
    rz = jax.nn.sigmoid(gi[:, :2 * H] + gh[:, :2 * H] + brz[...])
    r = rz[:, :H]
    z = rz[:, H:]
    n = jnp.tanh(gi[:, 2 * H:] + bn_i[...]
                 + r * (gh[:, 2 * H:] + bn_h[...]))
    hn = jax.nn.relu((1.0 - z) * n + z * h)
    hn_ref[...] = hn
    mn_ref[...] = jnp.dot(hn, wnext[...], preferred_element_type=jnp.float32)


def _gru(h, agg, wihT, whhT, brz, bn_i, bn_h, wnext):
    per_core = HALF // BN
    return pl.pallas_call(
        _gru_body,
        grid=(GRID,),
        in_specs=[
            pl.BlockSpec((BN, H), lambda i: (i, 0)),
            pl.BlockSpec((1, BN, H), lambda i: (i // per_core, i % per_core, 0)),
            pl.BlockSpec((H, 3 * H), lambda i: (0, 0)),
            pl.BlockSpec((H, 3 * H), lambda i: (0, 0)),
            pl.BlockSpec((1, 2 * H), lambda i: (0, 0)),
            pl.BlockSpec((1, H), lambda i: (0, 0)),
            pl.BlockSpec((1, H), lambda i: (0, 0)),
            pl.BlockSpec((H, H), lambda i: (0, 0)),
        ],
        out_specs=[pl.BlockSpec((BN, H), lambda i: (i, 0))] * 2,
        out_shape=[jax.ShapeDtypeStruct((N, H), jnp.float32)] * 2,
    )(h, agg, wihT, whhT, brz, bn_i, bn_h, wnext)


def _gru_pool_body(h_ref, a_ref, wihT, whhT, brz, bn_i, bn_h,
                   batch_ref, f1w, f1b, f2w, f2b, f3w, f3b,
                   out_ref, sums, cnt):
    i = pl.program_id(0)

    @pl.when(i == 0)
    def _():
        sums[...] = jnp.zeros_like(sums)
        cnt[...] = jnp.zeros_like(cnt)

    h = h_ref[...]
    a = a_ref[0]
    gi = jnp.dot(a, wihT[...], preferred_element_type=jnp.float32)
    gh = jnp.dot(h, whhT[...], preferred_element_type=jnp.float32)
    rz = jax.nn.sigmoid(gi[:, :2 * H] + gh[:, :2 * H] + brz[...])
    r = rz[:, :H]
    z = rz[:, H:]
    n = jnp.tanh(gi[:, 2 * H:] + bn_i[...]
                 + r * (gh[:, 2 * H:] + bn_h[...]))
    hn = jax.nn.relu((1.0 - z) * n + z * h)

    onehot = (batch_ref[...] == lax.broadcasted_iota(jnp.int32, (1, NG), 1)
              ).astype(jnp.float32)
    sums[...] += lax.dot_general(onehot, hn, (((0,), (0,)), ((), ())),
                                 preferred_element_type=jnp.float32)
    cnt[...] += lax.dot_general(onehot, jnp.ones((BN, 1), jnp.float32),
                                (((0,), (0,)), ((), ())),
                                preferred_element_type=jnp.float32)

    @pl.when(i == GRID - 1)
    def _():
        pooled = sums[...] / jnp.maximum(cnt[...], 1.0)
        o = jax.nn.relu(jnp.dot(pooled, f1w[...],
                                preferred_element_type=jnp.float32) + f1b[...])
        o = jax.nn.relu(jnp.dot(o, f2w[...],
                                preferred_element_type=jnp.float32) + f2b[...])
        o = jnp.dot(o, f3w[...], preferred_element_type=jnp.float32) + f3b[...]
        out_ref[...] = o


def _gru_pool(h, agg, wihT, whhT, brz, bn_i, bn_h, batch2d,
              f1w, f1b, f2w, f2b, f3w, f3b):
    per_core = HALF // BN
    return pl.pallas_call(
        _gru_pool_body,
        grid=(GRID,),
        in_specs=[
            pl.BlockSpec((BN, H), lambda i: (i, 0)),
            pl.BlockSpec((1, BN, H), lambda i: (i // per_core, i % per_core, 0)),
            pl.BlockSpec((H, 3 * H), lambda i: (0, 0)),
            pl.BlockSpec((H, 3 * H), lambda i: (0, 0)),
            pl.BlockSpec((1, 2 * H), lambda i: (0, 0)),
            pl.BlockSpec((1, H), lambda i: (0, 0)),
            pl.BlockSpec((1, H), lambda i: (0, 0)),
            pl.BlockSpec((BN, 1), lambda i: (i, 0)),
            pl.BlockSpec((H, H // 2), lambda i: (0, 0)),
            pl.BlockSpec((1, H // 2), lambda i: (0, 0)),
            pl.BlockSpec((H // 2, H // 4), lambda i: (0, 0)),
            pl.BlockSpec((1, H // 4), lambda i: (0, 0)),
            pl.BlockSpec((H // 4, 1), lambda i: (0, 0)),
            pl.BlockSpec((1, 1), lambda i: (0, 0)),
        ],
        out_specs=pl.BlockSpec((NG, 1), lambda i: (0, 0)),
        out_shape=jax.ShapeDtypeStruct((NG, 1), jnp.float32),
        scratch_shapes=[
            pltpu.VMEM((NG, NG), jnp.float32),
            pltpu.VMEM((NG, 1), jnp.float32),
        ],
    )(h, agg, wihT, whhT, brz, bn_i, bn_h, batch2d,
      f1w, f1b, f2w, f2b, f3w, f3b)


def kernel(x, edge_index, edge_attr, batch, node_emb, edge_lin_w, edge_lin_b,
           conv_weight, gru_Wih, gru_Whh, gru_bih, gru_bhh,
           fc1_w, fc1_b, fc2_w, fc2_b, fc3_w, fc3_b):
    src = edge_index[0]
    dst = edge_index[1]
    pad = EPAD - E
    srcp = jnp.concatenate([src, jnp.zeros((pad,), jnp.int32)])
    dstp = jnp.concatenate([dst, jnp.full((pad,), -1, jnp.int32)])

    emb_pad = jnp.pad(node_emb, ((0, CODES - node_emb.shape[0]), (0, 0)))

    h, m = _pre(x, emb_pad, conv_weight[0])
    psrc, pdst, cnts = _sc_partition(srcp, dstp)

    for i in range(NUM_LAYERS):
        agg = _sc_scatter(m, psrc, pdst, cnts)
        wihT = gru_Wih[i].T
        whhT = gru_Whh[i].T
        brz = (gru_bih[i, :2 * H] + gru_bhh[i, :2 * H]).reshape(1, 2 * H)
        bn_i = gru_bih[i, 2 * H:].reshape(1, H)
        bn_h = gru_bhh[i, 2 * H:].reshape(1, H)
        if i + 1 < NUM_LAYERS:
            h, m = _gru(h, agg, wihT, whhT, brz, bn_i, bn_h,
                        conv_weight[i + 1])
        else:
            out = _gru_pool(h, agg, wihT, whhT, brz, bn_i, bn_h,
                            batch.reshape(N, 1),
                            fc1_w.T, fc1_b.reshape(1, H // 2),
                            fc2_w.T, fc2_b.reshape(1, H // 4),
                            fc3_w.T, fc3_b.reshape(1, 1))
    return out[:, 0]


# speedup vs baseline: 1.7007x; 1.6919x over previous
import jax
import jax.numpy as jnp
from jax import lax
from jax.experimental import pallas as pl
from jax.experimental.pallas import tpu as pltpu
from jax.experimental.pallas import tpu_sc as plsc

N = 50000
E = 800000
H = 64
NG = 64
NUM_LAYERS = 3

NCORES = 2
NSUB = 16
HALF = N // NCORES
CHUNK = 128
CHUNKS_PER_SUB = 400
EPAD = NSUB * CHUNKS_PER_SUB * CHUNK
ACC_ROWS = 25088
NW = 32
EPW = EPAD // NW
NBUCKET = 4
PGROUP = 1024
SLOTCAP = EPW + PGROUP
STAGECAP = 2 * PGROUP + PGROUP + 16
TRASH_OFF = 3 * PGROUP
NPHASE = 2
QUART = HALF // NPHASE
ACC2_ROWS = 25104
ZROWS = 128
ZPS = ACC2_ROWS // NSUB

BN = 5000
GRID = N // BN
CODES = 104


NSLOT = 2
GROUP = NSLOT * CHUNK
GROUPS_PER_SUB = CHUNKS_PER_SUB // NSLOT
ROWBYTES = CHUNK * H * 4


def _sc_scatter_body(m_hbm, src_hbm, dst_hbm, out_hbm, *scr):
    srcgs = scr[0:2]
    dstgs = scr[2:4]
    idxs = scr[4:4 + NSLOT]
    rows = scr[4 + NSLOT:4 + 2 * NSLOT]
    zrow_v = scr[4 + 2 * NSLOT]
    acc = scr[5 + 2 * NSLOT]
    gsems = scr[6 + 2 * NSLOT:6 + 3 * NSLOT]
    ssems = scr[6 + 3 * NSLOT:6 + 4 * NSLOT]
    pfsems = scr[6 + 4 * NSLOT:8 + 4 * NSLOT]
    c = lax.axis_index("c")
    s = lax.axis_index("s")

    zero16 = jnp.zeros((16,), jnp.float32)

    def zrow_body(i, carry):
        for k in range(H // 16):
            zrow_v[i, pl.ds(k * 16, 16)] = zero16
        return carry

    lax.fori_loop(0, ZROWS, zrow_body, 0)

    node_base = c * HALF
    zrow0 = s * ZPS
    for j in range(ZPS // ZROWS):
        pltpu.sync_copy(zrow_v, acc.at[pl.ds(zrow0 + j * ZROWS, ZROWS)])
    rem = ZPS % ZROWS
    if rem:
        pltpu.sync_copy(zrow_v.at[pl.ds(0, rem)],
                        acc.at[pl.ds(zrow0 + (ZPS // ZROWS) * ZROWS, rem)])

    plsc.subcore_barrier()

    def stage_base(o):
        return (s * GROUPS_PER_SUB + o) * GROUP

    def fire_stage(o, u):
        pltpu.async_copy(src_hbm.at[pl.ds(stage_base(o), GROUP)],
                         srcgs[u], pfsems[u])
        pltpu.async_copy(dst_hbm.at[pl.ds(stage_base(o), GROUP)],
                         dstgs[u], pfsems[u])

    def wait_stage(u):
        pltpu.make_async_copy(src_hbm.at[pl.ds(0, GROUP)], srcgs[u],
                              pfsems[u]).wait()
        pltpu.make_async_copy(dst_hbm.at[pl.ds(0, GROUP)], dstgs[u],
                              pfsems[u]).wait()

    fire_stage(0, 0)
    fire_stage(1, 1)

    def pair_body(t, carry):
        for u in range(2):
            o = 2 * t + u
            wait_stage(u)
            gathers = []
            for b in range(NSLOT):
                @pl.when(o > 0)
                def _(b=b):
                    pltpu.make_async_copy(rows[b], acc.at[idxs[b]],
                                          ssems[b]).wait()
                gathers.append(
                    pltpu.async_copy(
                        m_hbm.at[srcgs[u].at[pl.ds(b * CHUNK, CHUNK)]],
                        rows[b], gsems[b]))
            for b in range(NSLOT):
                for k in range(CHUNK // 16):
                    d = dstgs[u][pl.ds(b * CHUNK + k * 16, 16)]
                    loc = d - node_base
                    oob = (loc < 0) | (loc >= HALF)
                    idxs[b][pl.ds(k * 16, 16)] = jnp.where(oob, HALF, loc)
                gathers[b].wait()
                pltpu.async_copy(rows[b], acc.at[idxs[b]], ssems[b],
                                 add=True)
            @pl.when(o + 2 < GROUPS_PER_SUB)
            def _(o=o, u=u):
                fire_stage(o + 2, u)
        return carry

    lax.fori_loop(0, GROUPS_PER_SUB // 2, pair_body, 0)

    for b in range(NSLOT):
        pltpu.make_async_copy(rows[b], acc.at[idxs[b]], ssems[b]).wait()

    plsc.subcore_barrier()
    @pl.when(s < 8)
    def _():
        roff = s * 1563
        pltpu.sync_copy(acc.at[pl.ds(roff, 1563)],
                        out_hbm.at[c, pl.ds(roff, 1563)])

    @pl.when(s >= 8)
    def _():
        roff = 8 * 1563 + (s - 8) * 1562
        pltpu.sync_copy(acc.at[pl.ds(roff, 1562)],
                        out_hbm.at[c, pl.ds(roff, 1562)])


def _sc_scatter(m, srcp, dstp):
    mesh = plsc.VectorSubcoreMesh(core_axis_name="c", subcore_axis_name="s")
    f = pl.kernel(
        _sc_scatter_body,
        out_type=jax.ShapeDtypeStruct((NCORES, ACC_ROWS, H), jnp.float32),
        mesh=mesh,
        scratch_types=(
            [pltpu.VMEM((GROUP,), jnp.int32)] * 4
            + [pltpu.VMEM((CHUNK,), jnp.int32)] * NSLOT
            + [pltpu.VMEM((CHUNK, H), jnp.float32)] * NSLOT
            + [pltpu.VMEM((ZROWS, H), jnp.float32)]
            + [pltpu.VMEM_SHARED((ACC2_ROWS, H), jnp.float32)]
            + [pltpu.SemaphoreType.DMA] * (2 * NSLOT + 2)
        ),
        compiler_params=pltpu.CompilerParams(use_tc_tiling_on_sc=False,
                                             needs_layout_passes=False),
    )
    return f(m, srcp, dstp)


def _pre_body(x_ref, emb_ref, w1_ref, h0_ref, m1_ref):
    codes = x_ref[...]
    onehot = (codes == lax.broadcasted_iota(jnp.int32, (1, CODES), 1)
              ).astype(jnp.float32)
    h0 = lax.dot_general(onehot, emb_ref[...], (((1,), (0,)), ((), ())),
                         preferred_element_type=jnp.float32)
    h0_ref[...] = h0
    m1_ref[...] = jnp.dot(h0, w1_ref[...], preferred_element_type=jnp.float32)


def _pre(x, emb_pad, w1):
    return pl.pallas_call(
        _pre_body,
        grid=(GRID,),
        in_specs=[
            pl.BlockSpec((BN, 1), lambda i: (i, 0)),
            pl.BlockSpec((CODES, H), lambda i: (0, 0)),
            pl.BlockSpec((H, H), lambda i: (0, 0)),
        ],
        out_specs=[
            pl.BlockSpec((BN, H), lambda i: (i, 0)),
            pl.BlockSpec((BN, H), lambda i: (i, 0)),
        ],
        out_shape=[jax.ShapeDtypeStruct((N, H), jnp.float32)] * 2,
    )(x, emb_pad, w1)


def _gru_body(h_ref, a_ref, wihT, whhT, brz, bn_i, bn_h, wnext,
              hn_ref, mn_ref):
    h = h_ref[...]
    a = a_ref[0]
    gi = jnp.dot(a, wihT[...], preferred_element_type=jnp.float32)
    gh = jnp.dot(h, whhT[...], preferred_element_type=jnp.float32)
    rz = jax.nn.sigmoid(gi[:, :2 * H] + gh[:, :2 * H] + brz[...])
    r = rz[:, :H]
    z = rz[:, H:]
    n = jnp.tanh(gi[:, 2 * H:] + bn_i[...]
                 + r * (gh[:, 2 * H:] + bn_h[...]))
    hn = jax.nn.relu((1.0 - z) * n + z * h)
    hn_ref[...] = hn
    mn_ref[...] = jnp.dot(hn, wnext[...], preferred_element_type=jnp.float32)


def _gru(h, agg, wihT, whhT, brz, bn_i, bn_h, wnext):
    per_core = HALF // BN
    return pl.pallas_call(
        _gru_body,
        grid=(GRID,),
        in_specs=[
            pl.BlockSpec((BN, H), lambda i: (i, 0)),
            pl.BlockSpec((1, BN, H), lambda i: (i // per_core, i % per_core, 0)),
            pl.BlockSpec((H, 3 * H), lambda i: (0, 0)),
            pl.BlockSpec((H, 3 * H), lambda i: (0, 0)),
            pl.BlockSpec((1, 2 * H), lambda i: (0, 0)),
            pl.BlockSpec((1, H), lambda i: (0, 0)),
            pl.BlockSpec((1, H), lambda i: (0, 0)),
            pl.BlockSpec((H, H), lambda i: (0, 0)),
        ],
        out_specs=[pl.BlockSpec((BN, H), lambda i: (i, 0))] * 2,
        out_shape=[jax.ShapeDtypeStruct((N, H), jnp.float32)] * 2,
    )(h, agg, wihT, whhT, brz, bn_i, bn_h, wnext)


def _gru_pool_body(h_ref, a_ref, wihT, whhT, brz, bn_i, bn_h,
                   batch_ref, f1w, f1b, f2w, f2b, f3w, f3b,
                   out_ref, sums, cnt):
    i = pl.program_id(0)

    @pl.when(i == 0)
    def _():
        sums[...] = jnp.zeros_like(sums)
        cnt[...] = jnp.zeros_like(cnt)

    h = h_ref[...]
    a = a_ref[0]
    gi = jnp.dot(a, wihT[...], preferred_element_type=jnp.float32)
    gh = jnp.dot(h, whhT[...], preferred_element_type=jnp.float32)
    rz = jax.nn.sigmoid(gi[:, :2 * H] + gh[:, :2 * H] + brz[...])
    r = rz[:, :H]
    z = rz[:, H:]
    n = jnp.tanh(gi[:, 2 * H:] + bn_i[...]
                 + r * (gh[:, 2 * H:] + bn_h[...]))
    hn = jax.nn.relu((1.0 - z) * n + z * h)

    onehot = (batch_ref[...] == lax.broadcasted_iota(jnp.int32, (1, NG), 1)
              ).astype(jnp.float32)
    sums[...] += lax.dot_general(onehot, hn, (((0,), (0,)), ((), ())),
                                 preferred_element_type=jnp.float32)
    cnt[...] += lax.dot_general(onehot, jnp.ones((BN, 1), jnp.float32),
                                (((0,), (0,)), ((), ())),
                                preferred_element_type=jnp.float32)

    @pl.when(i == GRID - 1)
    def _():
        pooled = sums[...] / jnp.maximum(cnt[...], 1.0)
        o = jax.nn.relu(jnp.dot(pooled, f1w[...],
                                preferred_element_type=jnp.float32) + f1b[...])
        o = jax.nn.relu(jnp.dot(o, f2w[...],
                                preferred_element_type=jnp.float32) + f2b[...])
        o = jnp.dot(o, f3w[...], preferred_element_type=jnp.float32) + f3b[...]
        out_ref[...] = o


def _gru_pool(h, agg, wihT, whhT, brz, bn_i, bn_h, batch2d,
              f1w, f1b, f2w, f2b, f3w, f3b):
    per_core = HALF // BN
    return pl.pallas_call(
        _gru_pool_body,
        grid=(GRID,),
        in_specs=[
            pl.BlockSpec((BN, H), lambda i: (i, 0)),
            pl.BlockSpec((1, BN, H), lambda i: (i // per_core, i % per_core, 0)),
            pl.BlockSpec((H, 3 * H), lambda i: (0, 0)),
            pl.BlockSpec((H, 3 * H), lambda i: (0, 0)),
            pl.BlockSpec((1, 2 * H), lambda i: (0, 0)),
            pl.BlockSpec((1, H), lambda i: (0, 0)),
            pl.BlockSpec((1, H), lambda i: (0, 0)),
            pl.BlockSpec((BN, 1), lambda i: (i, 0)),
            pl.BlockSpec((H, H // 2), lambda i: (0, 0)),
            pl.BlockSpec((1, H // 2), lambda i: (0, 0)),
            pl.BlockSpec((H // 2, H // 4), lambda i: (0, 0)),
            pl.BlockSpec((1, H // 4), lambda i: (0, 0)),
            pl.BlockSpec((H // 4, 1), lambda i: (0, 0)),
            pl.BlockSpec((1, 1), lambda i: (0, 0)),
        ],
        out_specs=pl.BlockSpec((NG, 1), lambda i: (0, 0)),
        out_shape=jax.ShapeDtypeStruct((NG, 1), jnp.float32),
        scratch_shapes=[
            pltpu.VMEM((NG, NG), jnp.float32),
            pltpu.VMEM((NG, 1), jnp.float32),
        ],
    )(h, agg, wihT, whhT, brz, bn_i, bn_h, batch2d,
      f1w, f1b, f2w, f2b, f3w, f3b)


def kernel(x, edge_index, edge_attr, batch, node_emb, edge_lin_w, edge_lin_b,
           conv_weight, gru_Wih, gru_Whh, gru_bih, gru_bhh,
           fc1_w, fc1_b, fc2_w, fc2_b, fc3_w, fc3_b):
    src = edge_index[0]
    dst = edge_index[1]
    pad = EPAD - E
    srcp = jnp.concatenate([src, jnp.zeros((pad,), jnp.int32)])
    dstp = jnp.concatenate([dst, jnp.full((pad,), -1, jnp.int32)])

    emb_pad = jnp.pad(node_emb, ((0, CODES - node_emb.shape[0]), (0, 0)))

    h, m = _pre(x, emb_pad, conv_weight[0])

    for i in range(NUM_LAYERS):
        agg = _sc_scatter(m, srcp, dstp)
        wihT = gru_Wih[i].T
        whhT = gru_Whh[i].T
        brz = (gru_bih[i, :2 * H] + gru_bhh[i, :2 * H]).reshape(1, 2 * H)
        bn_i = gru_bih[i, 2 * H:].reshape(1, H)
        bn_h = gru_bhh[i, 2 * H:].reshape(1, H)
        if i + 1 < NUM_LAYERS:
            h, m = _gru(h, agg, wihT, whhT, brz, bn_i, bn_h,
                        conv_weight[i + 1])
        else:
            out = _gru_pool(h, agg, wihT, whhT, brz, bn_i, bn_h,
                            batch.reshape(N, 1),
                            fc1_w.T, fc1_b.reshape(1, H // 2),
                            fc2_w.T, fc2_b.reshape(1, H // 4),
                            fc3_w.T, fc3_b.reshape(1, 1))
    return out[:, 0]


# speedup vs baseline: 1.8423x; 1.0833x over previous
import jax
import jax.numpy as jnp
from jax import lax
from jax.experimental import pallas as pl
from jax.experimental.pallas import tpu as pltpu
from jax.experimental.pallas import tpu_sc as plsc

N = 50000
E = 800000
H = 64
NG = 64
NUM_LAYERS = 3

NCORES = 2
NSUB = 16
HALF = N // NCORES
CHUNK = 128
CHUNKS_PER_SUB = 400
EPAD = NSUB * CHUNKS_PER_SUB * CHUNK
ACC_ROWS = 25088
NW = 32
EPW = EPAD // NW
NBUCKET = 4
PGROUP = 1024
SLOTCAP = EPW + PGROUP
STAGECAP = 2 * PGROUP + PGROUP + 16
TRASH_OFF = 3 * PGROUP
NPHASE = 2
QUART = HALF // NPHASE
ACC2_ROWS = 25104
ZROWS = 128
ZPS = ACC2_ROWS // NSUB

BN = 5000
GRID = N // BN
CODES = 104


NSLOT = 2
GROUP = NSLOT * CHUNK
GROUPS_PER_SUB = CHUNKS_PER_SUB // NSLOT
ROWBYTES = CHUNK * H * 4


def _sc_scatter_body(m_hbm, src_hbm, dst_hbm, out_hbm, *scr):
    srcgs = scr[0:2]
    dstgs = scr[2:4]
    idxs = scr[4:4 + NSLOT]
    rows = scr[4 + NSLOT:4 + 2 * NSLOT]
    zrow_v = scr[4 + 2 * NSLOT]
    acc = scr[5 + 2 * NSLOT]
    gsems = scr[6 + 2 * NSLOT:6 + 3 * NSLOT]
    ssems = scr[6 + 3 * NSLOT:6 + 4 * NSLOT]
    pfsems = scr[6 + 4 * NSLOT:8 + 4 * NSLOT]
    c = lax.axis_index("c")
    s = lax.axis_index("s")

    zero16 = jnp.zeros((16,), jnp.float32)

    def zrow_body(i, carry):
        for k in range(H // 16):
            zrow_v[i, pl.ds(k * 16, 16)] = zero16
        return carry

    lax.fori_loop(0, ZROWS, zrow_body, 0)

    node_base = c * HALF
    zrow0 = s * ZPS
    for j in range(ZPS // ZROWS):
        pltpu.sync_copy(zrow_v, acc.at[pl.ds(zrow0 + j * ZROWS, ZROWS)])
    rem = ZPS % ZROWS
    if rem:
        pltpu.sync_copy(zrow_v.at[pl.ds(0, rem)],
                        acc.at[pl.ds(zrow0 + (ZPS // ZROWS) * ZROWS, rem)])

    plsc.subcore_barrier()

    def stage_base(o):
        return (s * GROUPS_PER_SUB + o) * GROUP

    def fire_stage(o, u):
        pltpu.async_copy(src_hbm.at[pl.ds(stage_base(o), GROUP)],
                         srcgs[u], pfsems[u])
        pltpu.async_copy(dst_hbm.at[pl.ds(stage_base(o), GROUP)],
                         dstgs[u], pfsems[u])

    def wait_stage(u):
        pltpu.make_async_copy(src_hbm.at[pl.ds(0, GROUP)], srcgs[u],
                              pfsems[u]).wait()
        pltpu.make_async_copy(dst_hbm.at[pl.ds(0, GROUP)], dstgs[u],
                              pfsems[u]).wait()

    fire_stage(0, 0)
    fire_stage(1, 1)

    def pair_body(t, carry):
        for u in range(2):
            o = 2 * t + u
            wait_stage(u)
            gathers = []
            for b in range(NSLOT):
                @pl.when(o > 0)
                def _(b=b):
                    pltpu.make_async_copy(rows[b], acc.at[idxs[b]],
                                          ssems[b]).wait()
                gathers.append(
                    pltpu.async_copy(
                        m_hbm.at[srcgs[u].at[pl.ds(b * CHUNK, CHUNK)]],
                        rows[b], gsems[b]))
            for b in range(NSLOT):
                for k in range(CHUNK // 16):
                    d = dstgs[u][pl.ds(b * CHUNK + k * 16, 16)]
                    loc = d - node_base
                    oob = (loc < 0) | (loc >= HALF)
                    idxs[b][pl.ds(k * 16, 16)] = jnp.where(oob, HALF + s, loc)
                gathers[b].wait()
                pltpu.async_copy(rows[b], acc.at[idxs[b]], ssems[b],
                                 add=True)
            @pl.when(o + 2 < GROUPS_PER_SUB)
            def _(o=o, u=u):
                fire_stage(o + 2, u)
        return carry

    lax.fori_loop(0, GROUPS_PER_SUB // 2, pair_body, 0)

    for b in range(NSLOT):
        pltpu.make_async_copy(rows[b], acc.at[idxs[b]], ssems[b]).wait()

    plsc.subcore_barrier()
    @pl.when(s < 8)
    def _():
        roff = s * 1563
        pltpu.sync_copy(acc.at[pl.ds(roff, 1563)],
                        out_hbm.at[c, pl.ds(roff, 1563)])

    @pl.when(s >= 8)
    def _():
        roff = 8 * 1563 + (s - 8) * 1562
        pltpu.sync_copy(acc.at[pl.ds(roff, 1562)],
                        out_hbm.at[c, pl.ds(roff, 1562)])


def _sc_scatter(m, srcp, dstp):
    mesh = plsc.VectorSubcoreMesh(core_axis_name="c", subcore_axis_name="s")
    f = pl.kernel(
        _sc_scatter_body,
        out_type=jax.ShapeDtypeStruct((NCORES, ACC_ROWS, H), jnp.float32),
        mesh=mesh,
        scratch_types=(
            [pltpu.VMEM((GROUP,), jnp.int32)] * 4
            + [pltpu.VMEM((CHUNK,), jnp.int32)] * NSLOT
            + [pltpu.VMEM((CHUNK, H), jnp.float32)] * NSLOT
            + [pltpu.VMEM((ZROWS, H), jnp.float32)]
            + [pltpu.VMEM_SHARED((ACC2_ROWS, H), jnp.float32)]
            + [pltpu.SemaphoreType.DMA] * (2 * NSLOT + 2)
        ),
        compiler_params=pltpu.CompilerParams(use_tc_tiling_on_sc=False,
                                             needs_layout_passes=False),
    )
    return f(m, srcp, dstp)


def _pre_body(x_ref, emb_ref, w1_ref, h0_ref, m1_ref):
    codes = x_ref[...]
    onehot = (codes == lax.broadcasted_iota(jnp.int32, (1, CODES), 1)
              ).astype(jnp.float32)
    h0 = lax.dot_general(onehot, emb_ref[...], (((1,), (0,)), ((), ())),
                         preferred_element_type=jnp.float32)
    h0_ref[...] = h0
    m1_ref[...] = jnp.dot(h0, w1_ref[...], preferred_element_type=jnp.float32)


def _pre(x, emb_pad, w1):
    return pl.pallas_call(
        _pre_body,
        grid=(GRID,),
        in_specs=[
            pl.BlockSpec((BN, 1), lambda i: (i, 0)),
            pl.BlockSpec((CODES, H), lambda i: (0, 0)),
            pl.BlockSpec((H, H), lambda i: (0, 0)),
        ],
        out_specs=[
            pl.BlockSpec((BN, H), lambda i: (i, 0)),
            pl.BlockSpec((BN, H), lambda i: (i, 0)),
        ],
        out_shape=[jax.ShapeDtypeStruct((N, H), jnp.float32)] * 2,
    )(x, emb_pad, w1)


def _gru_body(h_ref, a_ref, wihT, whhT, brz, bn_i, bn_h, wnext,
              hn_ref, mn_ref):
    h = h_ref[...]
    a = a_ref[0]
    gi = jnp.dot(a, wihT[...], preferred_element_type=jnp.float32)
    gh = jnp.dot(h, whhT[...], preferred_element_type=jnp.float32)
    rz = jax.nn.sigmoid(gi[:, :2 * H] + gh[:, :2 * H] + brz[...])
    r = rz[:, :H]
    z = rz[:, H:]
    n = jnp.tanh(gi[:, 2 * H:] + bn_i[...]
                 + r * (gh[:, 2 * H:] + bn_h[...]))
    hn = jax.nn.relu((1.0 - z) * n + z * h)
    hn_ref[...] = hn
    mn_ref[...] = jnp.dot(hn, wnext[...], preferred_element_type=jnp.float32)


def _gru(h, agg, wihT, whhT, brz, bn_i, bn_h, wnext):
    per_core = HALF // BN
    return pl.pallas_call(
        _gru_body,
        grid=(GRID,),
        in_specs=[
            pl.BlockSpec((BN, H), lambda i: (i, 0)),
            pl.BlockSpec((1, BN, H), lambda i: (i // per_core, i % per_core, 0)),
            pl.BlockSpec((H, 3 * H), lambda i: (0, 0)),
            pl.BlockSpec((H, 3 * H), lambda i: (0, 0)),
            pl.BlockSpec((1, 2 * H), lambda i: (0, 0)),
            pl.BlockSpec((1, H), lambda i: (0, 0)),
            pl.BlockSpec((1, H), lambda i: (0, 0)),
            pl.BlockSpec((H, H), lambda i: (0, 0)),
        ],
        out_specs=[pl.BlockSpec((BN, H), lambda i: (i, 0))] * 2,
        out_shape=[jax.ShapeDtypeStruct((N, H), jnp.float32)] * 2,
    )(h, agg, wihT, whhT, brz, bn_i, bn_h, wnext)


def _gru_pool_body(h_ref, a_ref, wihT, whhT, brz, bn_i, bn_h,
                   batch_ref, f1w, f1b, f2w, f2b, f3w, f3b,
                   out_ref, sums, cnt):
    i = pl.program_id(0)

    @pl.when(i == 0)
    def _():
        sums[...] = jnp.zeros_like(sums)
        cnt[...] = jnp.zeros_like(cnt)

    h = h_ref[...]
    a = a_ref[0]
    gi = jnp.dot(a, wihT[...], preferred_element_type=jnp.float32)
    gh = jnp.dot(h, whhT[...], preferred_element_type=jnp.float32)
    rz = jax.nn.sigmoid(gi[:, :2 * H] + gh[:, :2 * H] + brz[...])
    r = rz[:, :H]
    z = rz[:, H:]
    n = jnp.tanh(gi[:, 2 * H:] + bn_i[...]
                 + r * (gh[:, 2 * H:] + bn_h[...]))
    hn = jax.nn.relu((1.0 - z) * n + z * h)

    onehot = (batch_ref[...] == lax.broadcasted_iota(jnp.int32, (1, NG), 1)
              ).astype(jnp.float32)
    sums[...] += lax.dot_general(onehot, hn, (((0,), (0,)), ((), ())),
                                 preferred_element_type=jnp.float32)
    cnt[...] += lax.dot_general(onehot, jnp.ones((BN, 1), jnp.float32),
                                (((0,), (0,)), ((), ())),
                                preferred_element_type=jnp.float32)

    @pl.when(i == GRID - 1)
    def _():
        pooled = sums[...] / jnp.maximum(cnt[...], 1.0)
        o = jax.nn.relu(jnp.dot(pooled, f1w[...],
                                preferred_element_type=jnp.float32) + f1b[...])
        o = jax.nn.relu(jnp.dot(o, f2w[...],
                                preferred_element_type=jnp.float32) + f2b[...])
        o = jnp.dot(o, f3w[...], preferred_element_type=jnp.float32) + f3b[...]
        out_ref[...] = o


def _gru_pool(h, agg, wihT, whhT, brz, bn_i, bn_h, batch2d,
              f1w, f1b, f2w, f2b, f3w, f3b):
    per_core = HALF // BN
    return pl.pallas_call(
        _gru_pool_body,
        grid=(GRID,),
        in_specs=[
            pl.BlockSpec((BN, H), lambda i: (i, 0)),
            pl.BlockSpec((1, BN, H), lambda i: (i // per_core, i % per_core, 0)),
            pl.BlockSpec((H, 3 * H), lambda i: (0, 0)),
            pl.BlockSpec((H, 3 * H), lambda i: (0, 0)),
            pl.BlockSpec((1, 2 * H), lambda i: (0, 0)),
            pl.BlockSpec((1, H), lambda i: (0, 0)),
            pl.BlockSpec((1, H), lambda i: (0, 0)),
            pl.BlockSpec((BN, 1), lambda i: (i, 0)),
            pl.BlockSpec((H, H // 2), lambda i: (0, 0)),
            pl.BlockSpec((1, H // 2), lambda i: (0, 0)),
            pl.BlockSpec((H // 2, H // 4), lambda i: (0, 0)),
            pl.BlockSpec((1, H // 4), lambda i: (0, 0)),
            pl.BlockSpec((H // 4, 1), lambda i: (0, 0)),
            pl.BlockSpec((1, 1), lambda i: (0, 0)),
        ],
        out_specs=pl.BlockSpec((NG, 1), lambda i: (0, 0)),
        out_shape=jax.ShapeDtypeStruct((NG, 1), jnp.float32),
        scratch_shapes=[
            pltpu.VMEM((NG, NG), jnp.float32),
            pltpu.VMEM((NG, 1), jnp.float32),
        ],
    )(h, agg, wihT, whhT, brz, bn_i, bn_h, batch2d,
      f1w, f1b, f2w, f2b, f3w, f3b)


def kernel(x, edge_index, edge_attr, batch, node_emb, edge_lin_w, edge_lin_b,
           conv_weight, gru_Wih, gru_Whh, gru_bih, gru_bhh,
           fc1_w, fc1_b, fc2_w, fc2_b, fc3_w, fc3_b):
    src = edge_index[0]
    dst = edge_index[1]
    pad = EPAD - E
    srcp = jnp.concatenate([src, jnp.zeros((pad,), jnp.int32)])
    dstp = jnp.concatenate([dst, jnp.full((pad,), -1, jnp.int32)])

    emb_pad = jnp.pad(node_emb, ((0, CODES - node_emb.shape[0]), (0, 0)))

    h, m = _pre(x, emb_pad, conv_weight[0])

    for i in range(NUM_LAYERS):
        agg = _sc_scatter(m, srcp, dstp)
        wihT = gru_Wih[i].T
        whhT = gru_Whh[i].T
        brz = (gru_bih[i, :2 * H] + gru_bhh[i, :2 * H]).reshape(1, 2 * H)
        bn_i = gru_bih[i, 2 * H:].reshape(1, H)
        bn_h = gru_bhh[i, 2 * H:].reshape(1, H)
        if i + 1 < NUM_LAYERS:
            h, m = _gru(h, agg, wihT, whhT, brz, bn_i, bn_h,
                        conv_weight[i + 1])
        else:
            out = _gru_pool(h, agg, wihT, whhT, brz, bn_i, bn_h,
                            batch.reshape(N, 1),
                            fc1_w.T, fc1_b.reshape(1, H // 2),
                            fc2_w.T, fc2_b.reshape(1, H // 4),
                            fc3_w.T, fc3_b.reshape(1, 1))
    return out[:, 0]


# speedup vs baseline: 1.8629x; 1.0112x over previous
import jax
import jax.numpy as jnp
from jax import lax
from jax.experimental import pallas as pl
from jax.experimental.pallas import tpu as pltpu
from jax.experimental.pallas import tpu_sc as plsc

N = 50000
E = 800000
H = 64
NG = 64
NUM_LAYERS = 3

NCORES = 2
NSUB = 16
HALF = N // NCORES
CHUNK = 160
CHUNKS_PER_SUB = 320
EPAD = NSUB * CHUNKS_PER_SUB * CHUNK
ACC_ROWS = 25088
NW = 32
EPW = EPAD // NW
NBUCKET = 4
PGROUP = 1024
SLOTCAP = EPW + PGROUP
STAGECAP = 2 * PGROUP + PGROUP + 16
TRASH_OFF = 3 * PGROUP
NPHASE = 2
QUART = HALF // NPHASE
ACC2_ROWS = 25104
ZROWS = 128
ZPS = ACC2_ROWS // NSUB

BN = 5000
GRID = N // BN
CODES = 104


NSLOT = 2
GROUP = NSLOT * CHUNK
GROUPS_PER_SUB = CHUNKS_PER_SUB // NSLOT
ROWBYTES = CHUNK * H * 4


def _sc_scatter_body(m_hbm, src_hbm, dst_hbm, out_hbm, *scr):
    srcgs = scr[0:2]
    dstgs = scr[2:4]
    idxs = scr[4:4 + NSLOT]
    rows = scr[4 + NSLOT:4 + 2 * NSLOT]
    zrow_v = scr[4 + 2 * NSLOT]
    acc = scr[5 + 2 * NSLOT]
    gsems = scr[6 + 2 * NSLOT:6 + 3 * NSLOT]
    ssems = scr[6 + 3 * NSLOT:6 + 4 * NSLOT]
    pfsems = scr[6 + 4 * NSLOT:8 + 4 * NSLOT]
    c = lax.axis_index("c")
    s = lax.axis_index("s")

    zero16 = jnp.zeros((16,), jnp.float32)

    def zrow_body(i, carry):
        for k in range(H // 16):
            zrow_v[i, pl.ds(k * 16, 16)] = zero16
        return carry

    lax.fori_loop(0, ZROWS, zrow_body, 0)

    node_base = c * HALF
    zrow0 = s * ZPS
    for j in range(ZPS // ZROWS):
        pltpu.sync_copy(zrow_v, acc.at[pl.ds(zrow0 + j * ZROWS, ZROWS)])
    rem = ZPS % ZROWS
    if rem:
        pltpu.sync_copy(zrow_v.at[pl.ds(0, rem)],
                        acc.at[pl.ds(zrow0 + (ZPS // ZROWS) * ZROWS, rem)])

    plsc.subcore_barrier()

    def stage_base(o):
        return (s * GROUPS_PER_SUB + o) * GROUP

    def fire_stage(o, u):
        pltpu.async_copy(src_hbm.at[pl.ds(stage_base(o), GROUP)],
                         srcgs[u], pfsems[u])
        pltpu.async_copy(dst_hbm.at[pl.ds(stage_base(o), GROUP)],
                         dstgs[u], pfsems[u])

    def wait_stage(u):
        pltpu.make_async_copy(src_hbm.at[pl.ds(0, GROUP)], srcgs[u],
                              pfsems[u]).wait()
        pltpu.make_async_copy(dst_hbm.at[pl.ds(0, GROUP)], dstgs[u],
                              pfsems[u]).wait()

    fire_stage(0, 0)
    fire_stage(1, 1)

    def pair_body(t, carry):
        for u in range(2):
            o = 2 * t + u
            wait_stage(u)
            gathers = []
            for b in range(NSLOT):
                @pl.when(o > 0)
                def _(b=b):
                    pltpu.make_async_copy(rows[b], acc.at[idxs[b]],
                                          ssems[b]).wait()
                gathers.append(
                    pltpu.async_copy(
                        m_hbm.at[srcgs[u].at[pl.ds(b * CHUNK, CHUNK)]],
                        rows[b], gsems[b]))
            for b in range(NSLOT):
                for k in range(CHUNK // 16):
                    d = dstgs[u][pl.ds(b * CHUNK + k * 16, 16)]
                    loc = d - node_base
                    oob = (loc < 0) | (loc >= HALF)
                    idxs[b][pl.ds(k * 16, 16)] = jnp.where(oob, HALF + s, loc)
                gathers[b].wait()
                pltpu.async_copy(rows[b], acc.at[idxs[b]], ssems[b],
                                 add=True)
            @pl.when(o + 2 < GROUPS_PER_SUB)
            def _(o=o, u=u):
                fire_stage(o + 2, u)
        return carry

    lax.fori_loop(0, GROUPS_PER_SUB // 2, pair_body, 0)

    for b in range(NSLOT):
        pltpu.make_async_copy(rows[b], acc.at[idxs[b]], ssems[b]).wait()

    plsc.subcore_barrier()
    @pl.when(s < 8)
    def _():
        roff = s * 1563
        pltpu.sync_copy(acc.at[pl.ds(roff, 1563)],
                        out_hbm.at[c, pl.ds(roff, 1563)])

    @pl.when(s >= 8)
    def _():
        roff = 8 * 1563 + (s - 8) * 1562
        pltpu.sync_copy(acc.at[pl.ds(roff, 1562)],
                        out_hbm.at[c, pl.ds(roff, 1562)])


def _sc_scatter(m, srcp, dstp):
    mesh = plsc.VectorSubcoreMesh(core_axis_name="c", subcore_axis_name="s")
    f = pl.kernel(
        _sc_scatter_body,
        out_type=jax.ShapeDtypeStruct((NCORES, ACC_ROWS, H), jnp.float32),
        mesh=mesh,
        scratch_types=(
            [pltpu.VMEM((GROUP,), jnp.int32)] * 4
            + [pltpu.VMEM((CHUNK,), jnp.int32)] * NSLOT
            + [pltpu.VMEM((CHUNK, H), jnp.float32)] * NSLOT
            + [pltpu.VMEM((ZROWS, H), jnp.float32)]
            + [pltpu.VMEM_SHARED((ACC2_ROWS, H), jnp.float32)]
            + [pltpu.SemaphoreType.DMA] * (2 * NSLOT + 2)
        ),
        compiler_params=pltpu.CompilerParams(use_tc_tiling_on_sc=False,
                                             needs_layout_passes=False),
    )
    return f(m, srcp, dstp)


def _pre_body(x_ref, emb_ref, w1_ref, h0_ref, m1_ref):
    codes = x_ref[...]
    onehot = (codes == lax.broadcasted_iota(jnp.int32, (1, CODES), 1)
              ).astype(jnp.float32)
    h0 = lax.dot_general(onehot, emb_ref[...], (((1,), (0,)), ((), ())),
                         preferred_element_type=jnp.float32)
    h0_ref[...] = h0
    m1_ref[...] = jnp.dot(h0, w1_ref[...], preferred_element_type=jnp.float32)


def _pre(x, emb_pad, w1):
    return pl.pallas_call(
        _pre_body,
        grid=(GRID,),
        in_specs=[
            pl.BlockSpec((BN, 1), lambda i: (i, 0)),
            pl.BlockSpec((CODES, H), lambda i: (0, 0)),
            pl.BlockSpec((H, H), lambda i: (0, 0)),
        ],
        out_specs=[
            pl.BlockSpec((BN, H), lambda i: (i, 0)),
            pl.BlockSpec((BN, H), lambda i: (i, 0)),
        ],
        out_shape=[jax.ShapeDtypeStruct((N, H), jnp.float32)] * 2,
    )(x, emb_pad, w1)


def _gru_body(h_ref, a_ref, wihT, whhT, brz, bn_i, bn_h, wnext,
              hn_ref, mn_ref):
    h = h_ref[...]
    a = a_ref[0]
    gi = jnp.dot(a, wihT[...], preferred_element_type=jnp.float32)
    gh = jnp.dot(h, whhT[...], preferred_element_type=jnp.float32)
    rz = jax.nn.sigmoid(gi[:, :2 * H] + gh[:, :2 * H] + brz[...])
    r = rz[:, :H]
    z = rz[:, H:]
    n = jnp.tanh(gi[:, 2 * H:] + bn_i[...]
                 + r * (gh[:, 2 * H:] + bn_h[...]))
    hn = jax.nn.relu((1.0 - z) * n + z * h)
    hn_ref[...] = hn
    mn_ref[...] = jnp.dot(hn, wnext[...], preferred_element_type=jnp.float32)


def _gru(h, agg, wihT, whhT, brz, bn_i, bn_h, wnext):
    per_core = HALF // BN
    return pl.pallas_call(
        _gru_body,
        grid=(GRID,),
        in_specs=[
            pl.BlockSpec((BN, H), lambda i: (i, 0)),
            pl.BlockSpec((1, BN, H), lambda i: (i // per_core, i % per_core, 0)),
            pl.BlockSpec((H, 3 * H), lambda i: (0, 0)),
            pl.BlockSpec((H, 3 * H), lambda i: (0, 0)),
            pl.BlockSpec((1, 2 * H), lambda i: (0, 0)),
            pl.BlockSpec((1, H), lambda i: (0, 0)),
            pl.BlockSpec((1, H), lambda i: (0, 0)),
            pl.BlockSpec((H, H), lambda i: (0, 0)),
        ],
        out_specs=[pl.BlockSpec((BN, H), lambda i: (i, 0))] * 2,
        out_shape=[jax.ShapeDtypeStruct((N, H), jnp.float32)] * 2,
    )(h, agg, wihT, whhT, brz, bn_i, bn_h, wnext)


def _gru_pool_body(h_ref, a_ref, wihT, whhT, brz, bn_i, bn_h,
                   batch_ref, f1w, f1b, f2w, f2b, f3w, f3b,
                   out_ref, sums, cnt):
    i = pl.program_id(0)

    @pl.when(i == 0)
    def _():
        sums[...] = jnp.zeros_like(sums)
        cnt[...] = jnp.zeros_like(cnt)

    h = h_ref[...]
    a = a_ref[0]
    gi = jnp.dot(a, wihT[...], preferred_element_type=jnp.float32)
    gh = jnp.dot(h, whhT[...], preferred_element_type=jnp.float32)
    rz = jax.nn.sigmoid(gi[:, :2 * H] + gh[:, :2 * H] + brz[...])
    r = rz[:, :H]
    z = rz[:, H:]
    n = jnp.tanh(gi[:, 2 * H:] + bn_i[...]
                 + r * (gh[:, 2 * H:] + bn_h[...]))
    hn = jax.nn.relu((1.0 - z) * n + z * h)

    onehot = (batch_ref[...] == lax.broadcasted_iota(jnp.int32, (1, NG), 1)
              ).astype(jnp.float32)
    sums[...] += lax.dot_general(onehot, hn, (((0,), (0,)), ((), ())),
                                 preferred_element_type=jnp.float32)
    cnt[...] += lax.dot_general(onehot, jnp.ones((BN, 1), jnp.float32),
                                (((0,), (0,)), ((), ())),
                                preferred_element_type=jnp.float32)

    @pl.when(i == GRID - 1)
    def _():
        pooled = sums[...] / jnp.maximum(cnt[...], 1.0)
        o = jax.nn.relu(jnp.dot(pooled, f1w[...],
                                preferred_element_type=jnp.float32) + f1b[...])
        o = jax.nn.relu(jnp.dot(o, f2w[...],
                                preferred_element_type=jnp.float32) + f2b[...])
        o = jnp.dot(o, f3w[...], preferred_element_type=jnp.float32) + f3b[...]
        out_ref[...] = o


def _gru_pool(h, agg, wihT, whhT, brz, bn_i, bn_h, batch2d,
              f1w, f1b, f2w, f2b, f3w, f3b):
    per_core = HALF // BN
    return pl.pallas_call(
        _gru_pool_body,
        grid=(GRID,),
        in_specs=[
            pl.BlockSpec((BN, H), lambda i: (i, 0)),
            pl.BlockSpec((1, BN, H), lambda i: (i // per_core, i % per_core, 0)),
            pl.BlockSpec((H, 3 * H), lambda i: (0, 0)),
            pl.BlockSpec((H, 3 * H), lambda i: (0, 0)),
            pl.BlockSpec((1, 2 * H), lambda i: (0, 0)),
            pl.BlockSpec((1, H), lambda i: (0, 0)),
            pl.BlockSpec((1, H), lambda i: (0, 0)),
            pl.BlockSpec((BN, 1), lambda i: (i, 0)),
            pl.BlockSpec((H, H // 2), lambda i: (0, 0)),
            pl.BlockSpec((1, H // 2), lambda i: (0, 0)),
            pl.BlockSpec((H // 2, H // 4), lambda i: (0, 0)),
            pl.BlockSpec((1, H // 4), lambda i: (0, 0)),
            pl.BlockSpec((H // 4, 1), lambda i: (0, 0)),
            pl.BlockSpec((1, 1), lambda i: (0, 0)),
        ],
        out_specs=pl.BlockSpec((NG, 1), lambda i: (0, 0)),
        out_shape=jax.ShapeDtypeStruct((NG, 1), jnp.float32),
        scratch_shapes=[
            pltpu.VMEM((NG, NG), jnp.float32),
            pltpu.VMEM((NG, 1), jnp.float32),
        ],
    )(h, agg, wihT, whhT, brz, bn_i, bn_h, batch2d,
      f1w, f1b, f2w, f2b, f3w, f3b)


def kernel(x, edge_index, edge_attr, batch, node_emb, edge_lin_w, edge_lin_b,
           conv_weight, gru_Wih, gru_Whh, gru_bih, gru_bhh,
           fc1_w, fc1_b, fc2_w, fc2_b, fc3_w, fc3_b):
    src = edge_index[0]
    dst = edge_index[1]
    pad = EPAD - E
    srcp = jnp.concatenate([src, jnp.zeros((pad,), jnp.int32)])
    dstp = jnp.concatenate([dst, jnp.full((pad,), -1, jnp.int32)])

    emb_pad = jnp.pad(node_emb, ((0, CODES - node_emb.shape[0]), (0, 0)))

    h, m = _pre(x, emb_pad, conv_weight[0])

    for i in range(NUM_LAYERS):
        agg = _sc_scatter(m, srcp, dstp)
        wihT = gru_Wih[i].T
        whhT = gru_Whh[i].T
        brz = (gru_bih[i, :2 * H] + gru_bhh[i, :2 * H]).reshape(1, 2 * H)
        bn_i = gru_bih[i, 2 * H:].reshape(1, H)
        bn_h = gru_bhh[i, 2 * H:].reshape(1, H)
        if i + 1 < NUM_LAYERS:
            h, m = _gru(h, agg, wihT, whhT, brz, bn_i, bn_h,
                        conv_weight[i + 1])
        else:
            out = _gru_pool(h, agg, wihT, whhT, brz, bn_i, bn_h,
                            batch.reshape(N, 1),
                            fc1_w.T, fc1_b.reshape(1, H // 2),
                            fc2_w.T, fc2_b.reshape(1, H // 4),
                            fc3_w.T, fc3_b.reshape(1, 1))
    return out[:, 0]
